# Initial kernel scaffold; baseline (speedup 1.0000x reference)
import functools

import jax
import jax.numpy as jnp
import numpy as np
from jax import lax
from jax.experimental import pallas as pl
from jax.experimental.pallas import tpu as pltpu
from jax.experimental.pallas import tpu_sc as plsc

N_NODES = 10000
N_EDGES = 320000
D = 128
N_GROUPS = 8
GROUP_SIZE = 16
EPS = 1e-5

NC = 2
NS = 16
NW = NC * NS
EDGES_PER_WORKER = N_EDGES // NW
EBLK = 80
NBLK = EDGES_PER_WORKER // EBLK
ROWS_PER_SUB = N_NODES // NS

_HI = lax.Precision.HIGHEST

_GM_NP = np.kron(np.eye(N_GROUPS, dtype=np.float32),
                 np.ones((GROUP_SIZE, GROUP_SIZE), dtype=np.float32)) / GROUP_SIZE


def _pq_body(x_ref, wd_ref, ws_ref, p_ref, q_ref):
    xv = x_ref[...]
    p_ref[...] = jnp.dot(xv, wd_ref[...], precision=_HI)
    q_ref[...] = jnp.dot(xv, ws_ref[...], precision=_HI)


def _pq_call(x, wd, ws):
    blk = 400
    grid = N_NODES // blk
    return pl.pallas_call(
        _pq_body,
        grid=(grid,),
        in_specs=[
            pl.BlockSpec((blk, D), lambda i: (i, 0)),
            pl.BlockSpec((D, D), lambda i: (0, 0)),
            pl.BlockSpec((D, D), lambda i: (0, 0)),
        ],
        out_specs=[
            pl.BlockSpec((blk, D), lambda i: (i, 0)),
            pl.BlockSpec((blk, D), lambda i: (i, 0)),
        ],
        out_shape=[
            jax.ShapeDtypeStruct((N_NODES, D), jnp.float32),
            jax.ShapeDtypeStruct((N_NODES, D), jnp.float32),
        ],
    )(x, wd, ws)


def _gather_kernel(p_hbm, q_hbm, dst_hbm, src_hbm, pd_hbm, qs_hbm,
                   di_v, si_v, pbuf, qbuf, sem1, sem2):
    c = lax.axis_index("c")
    s = lax.axis_index("s")
    wid = s * NC + c
    base = wid * EDGES_PER_WORKER

    @pl.loop(0, NBLK)
    def _(j):
        off = pl.multiple_of(base + j * EBLK, 8)
        pltpu.sync_copy(dst_hbm.at[pl.ds(off, EBLK)], di_v)
        pltpu.sync_copy(src_hbm.at[pl.ds(off, EBLK)], si_v)
        cp1 = pltpu.async_copy(p_hbm.at[di_v], pbuf, sem1)
        cp2 = pltpu.async_copy(q_hbm.at[si_v], qbuf, sem2)
        cp1.wait()
        cp2.wait()
        pltpu.sync_copy(pbuf, pd_hbm.at[pl.ds(off, EBLK)])
        pltpu.sync_copy(qbuf, qs_hbm.at[pl.ds(off, EBLK)])


def _gather_call(p, q, dst, src):
    mesh = plsc.VectorSubcoreMesh(core_axis_name="c", subcore_axis_name="s")
    f = pl.kernel(
        _gather_kernel,
        mesh=mesh,
        out_type=(
            jax.ShapeDtypeStruct((N_EDGES, D), jnp.float32),
            jax.ShapeDtypeStruct((N_EDGES, D), jnp.float32),
        ),
        scratch_types=[
            pltpu.VMEM((EBLK,), jnp.int32),
            pltpu.VMEM((EBLK,), jnp.int32),
            pltpu.VMEM((EBLK, D), jnp.float32),
            pltpu.VMEM((EBLK, D), jnp.float32),
            pltpu.SemaphoreType.DMA,
            pltpu.SemaphoreType.DMA,
        ],
    )
    return f(p, q, dst, src)


def _mlp_body(pd_ref, qs_ref, ea_ref, w1e_ref, b1_ref, g_ref, bt_ref, gm_ref,
              a_ref):
    h = (pd_ref[...] + qs_ref[...]
         + jnp.dot(ea_ref[...], w1e_ref[...], precision=_HI) + b1_ref[...])
    gm = gm_ref[...]
    m = jnp.dot(h, gm, precision=_HI)
    sq = jnp.dot(h * h, gm, precision=_HI)
    var = sq - m * m
    y = (h - m) * lax.rsqrt(var + EPS) * g_ref[...] + bt_ref[...]
    a_ref[...] = y * jax.nn.sigmoid(y)


def _mlp_call(pd, qs, ea, w1e, b1, gamma, beta, gmat):
    blk = 1600
    grid = N_EDGES // blk
    return pl.pallas_call(
        _mlp_body,
        grid=(grid,),
        in_specs=[
            pl.BlockSpec((blk, D), lambda i: (i, 0)),
            pl.BlockSpec((blk, D), lambda i: (i, 0)),
            pl.BlockSpec((blk, 4), lambda i: (i, 0)),
            pl.BlockSpec((4, D), lambda i: (0, 0)),
            pl.BlockSpec((1, D), lambda i: (0, 0)),
            pl.BlockSpec((1, D), lambda i: (0, 0)),
            pl.BlockSpec((1, D), lambda i: (0, 0)),
            pl.BlockSpec((D, D), lambda i: (0, 0)),
        ],
        out_specs=pl.BlockSpec((blk, D), lambda i: (i, 0)),
        out_shape=jax.ShapeDtypeStruct((N_EDGES, D), jnp.float32),
    )(pd, qs, ea, w1e, b1, gamma, beta, gmat)


def _scatter_kernel(a_hbm, dst_hbm, zrow_hbm, zcnt_hbm, ones_hbm,
                    sp_hbm, cp_hbm,
                    idx_v, abuf, onesbuf, s_sh, c_sh, sem):
    c = lax.axis_index("c")
    s = lax.axis_index("s")
    wid = s * NC + c
    base = wid * EDGES_PER_WORKER
    rbase = pl.multiple_of(s * ROWS_PER_SUB, 8)

    pltpu.sync_copy(zrow_hbm, s_sh.at[pl.ds(rbase, ROWS_PER_SUB)])
    pltpu.sync_copy(zcnt_hbm, c_sh.at[pl.ds(rbase, ROWS_PER_SUB)])
    pltpu.sync_copy(ones_hbm, onesbuf)
    plsc.subcore_barrier()

    @pl.loop(0, NBLK)
    def _(j):
        off = pl.multiple_of(base + j * EBLK, 8)
        pltpu.sync_copy(a_hbm.at[pl.ds(off, EBLK)], abuf)
        pltpu.sync_copy(dst_hbm.at[pl.ds(off, EBLK)], idx_v)
        pltpu.sync_copy(abuf, s_sh.at[idx_v], add=True)
        pltpu.sync_copy(onesbuf, c_sh.at[idx_v], add=True)

    plsc.subcore_barrier()
    pltpu.sync_copy(s_sh.at[pl.ds(rbase, ROWS_PER_SUB)],
                    sp_hbm.at[c, pl.ds(rbase, ROWS_PER_SUB)])
    pltpu.sync_copy(c_sh.at[pl.ds(rbase, ROWS_PER_SUB)],
                    cp_hbm.at[c, pl.ds(rbase, ROWS_PER_SUB)])


def _scatter_call(a, dst, zrow, zcnt, ones):
    mesh = plsc.VectorSubcoreMesh(core_axis_name="c", subcore_axis_name="s")
    f = pl.kernel(
        _scatter_kernel,
        mesh=mesh,
        out_type=(
            jax.ShapeDtypeStruct((NC, N_NODES, D), jnp.float32),
            jax.ShapeDtypeStruct((NC, N_NODES, GROUP_SIZE), jnp.float32),
        ),
        scratch_types=[
            pltpu.VMEM((EBLK,), jnp.int32),
            pltpu.VMEM((EBLK, D), jnp.float32),
            pltpu.VMEM((EBLK, GROUP_SIZE), jnp.float32),
            pltpu.VMEM_SHARED((N_NODES, D), jnp.float32),
            pltpu.VMEM_SHARED((N_NODES, GROUP_SIZE), jnp.float32),
            pltpu.SemaphoreType.DMA,
        ],
    )
    return f(a, dst, zrow, zcnt, ones)


def _out_body(sp_ref, cp_ref, w2_ref, b2_ref, o_ref):
    sv = sp_ref[0] + sp_ref[1]
    cnt = cp_ref[0, :, 0:1] + cp_ref[1, :, 0:1]
    msg = jnp.dot(sv, w2_ref[...], precision=_HI) + cnt * b2_ref[...]
    o_ref[...] = msg / jnp.maximum(cnt, 1.0)


def _out_call(sp, cp, w2, b2):
    blk = 400
    grid = N_NODES // blk
    return pl.pallas_call(
        _out_body,
        grid=(grid,),
        in_specs=[
            pl.BlockSpec((NC, blk, D), lambda i: (0, i, 0)),
            pl.BlockSpec((NC, blk, GROUP_SIZE), lambda i: (0, i, 0)),
            pl.BlockSpec((D, D), lambda i: (0, 0)),
            pl.BlockSpec((1, D), lambda i: (0, 0)),
        ],
        out_specs=pl.BlockSpec((blk, D), lambda i: (i, 0)),
        out_shape=jax.ShapeDtypeStruct((N_NODES, D), jnp.float32),
    )(sp, cp, w2, b2)


def kernel(x, edge_index, edge_attr, W1, b1, gamma, beta, W2, b2):
    src = edge_index[0]
    dst = edge_index[1]
    w1d = W1[0:D]
    w1s = W1[D:2 * D]
    w1e = W1[2 * D:]
    gmat = jnp.asarray(_GM_NP)
    zrow = jnp.zeros((ROWS_PER_SUB, D), jnp.float32)
    zcnt = jnp.zeros((ROWS_PER_SUB, GROUP_SIZE), jnp.float32)
    ones = jnp.ones((EBLK, GROUP_SIZE), jnp.float32)

    p, q = _pq_call(x, w1d, w1s)
    pd, qs = _gather_call(p, q, dst, src)
    a = _mlp_call(pd, qs, edge_attr, w1e,
                  b1.reshape(1, D), gamma.reshape(1, D), beta.reshape(1, D),
                  gmat)
    sp, cp = _scatter_call(a, dst, zrow, zcnt, ones)
    return _out_call(sp, cp, W2, b2.reshape(1, D))


# speedup vs baseline: 2.7219x; 2.7219x over previous
import functools

import jax
import jax.numpy as jnp
import numpy as np
from jax import lax
from jax.experimental import pallas as pl
from jax.experimental.pallas import tpu as pltpu
from jax.experimental.pallas import tpu_sc as plsc

N_NODES = 10000
N_EDGES = 320000
D = 128
N_GROUPS = 8
GROUP_SIZE = 16
EPS = 1e-5

NC = 2
NS = 16
NW = NC * NS
EDGES_PER_WORKER = N_EDGES // NW
EBLK = 80
NBLK = EDGES_PER_WORKER // EBLK
N_PAD = 10240
ROWS_PER_SUB = N_PAD // NS

_HI = lax.Precision.HIGHEST

_GM_NP = np.kron(np.eye(N_GROUPS, dtype=np.float32),
                 np.ones((GROUP_SIZE, GROUP_SIZE), dtype=np.float32)) / GROUP_SIZE


def _pq_body(x_ref, wd_ref, ws_ref, p_ref, q_ref):
    xv = x_ref[...]
    p_ref[...] = jnp.dot(xv, wd_ref[...], precision=_HI)
    q_ref[...] = jnp.dot(xv, ws_ref[...], precision=_HI)


def _pq_call(x, wd, ws):
    blk = 400
    grid = N_NODES // blk
    return pl.pallas_call(
        _pq_body,
        grid=(grid,),
        in_specs=[
            pl.BlockSpec((blk, D), lambda i: (i, 0)),
            pl.BlockSpec((D, D), lambda i: (0, 0)),
            pl.BlockSpec((D, D), lambda i: (0, 0)),
        ],
        out_specs=[
            pl.BlockSpec((blk, D), lambda i: (i, 0)),
            pl.BlockSpec((blk, D), lambda i: (i, 0)),
        ],
        out_shape=[
            jax.ShapeDtypeStruct((N_NODES, D), jnp.float32),
            jax.ShapeDtypeStruct((N_NODES, D), jnp.float32),
        ],
    )(x, wd, ws)


def _gather_kernel(p_hbm, q_hbm, dst_hbm, src_hbm, pd_hbm, qs_hbm,
                   di_v, si_v, pbuf, qbuf, sem1, sem2):
    c = lax.axis_index("c")
    s = lax.axis_index("s")
    wid = s * NC + c
    base = wid * EDGES_PER_WORKER

    @pl.loop(0, NBLK)
    def _(j):
        off = pl.multiple_of(base + j * EBLK, 8)
        pltpu.sync_copy(dst_hbm.at[pl.ds(off, EBLK)], di_v)
        pltpu.sync_copy(src_hbm.at[pl.ds(off, EBLK)], si_v)
        cp1 = pltpu.async_copy(p_hbm.at[di_v], pbuf, sem1)
        cp2 = pltpu.async_copy(q_hbm.at[si_v], qbuf, sem2)
        cp1.wait()
        cp2.wait()
        pltpu.sync_copy(pbuf, pd_hbm.at[pl.ds(off, EBLK)])
        pltpu.sync_copy(qbuf, qs_hbm.at[pl.ds(off, EBLK)])


def _gather_call(p, q, dst, src):
    mesh = plsc.VectorSubcoreMesh(core_axis_name="c", subcore_axis_name="s")
    f = pl.kernel(
        _gather_kernel,
        mesh=mesh,
        out_type=(
            jax.ShapeDtypeStruct((N_EDGES, D), jnp.float32),
            jax.ShapeDtypeStruct((N_EDGES, D), jnp.float32),
        ),
        scratch_types=[
            pltpu.VMEM((EBLK,), jnp.int32),
            pltpu.VMEM((EBLK,), jnp.int32),
            pltpu.VMEM((EBLK, D), jnp.float32),
            pltpu.VMEM((EBLK, D), jnp.float32),
            pltpu.SemaphoreType.DMA,
            pltpu.SemaphoreType.DMA,
        ],
    )
    return f(p, q, dst, src)


def _mlp_body(pd_ref, qs_ref, ea_ref, w1e_ref, b1_ref, g_ref, bt_ref, gm_ref,
              a_ref):
    h = (pd_ref[...] + qs_ref[...]
         + jnp.dot(ea_ref[...], w1e_ref[...], precision=_HI) + b1_ref[...])
    gm = gm_ref[...]
    m = jnp.dot(h, gm, precision=_HI)
    sq = jnp.dot(h * h, gm, precision=_HI)
    var = sq - m * m
    y = (h - m) * lax.rsqrt(var + EPS) * g_ref[...] + bt_ref[...]
    a_ref[...] = y * jax.nn.sigmoid(y)


def _mlp_call(pd, qs, ea, w1e, b1, gamma, beta, gmat):
    blk = 1600
    grid = N_EDGES // blk
    return pl.pallas_call(
        _mlp_body,
        grid=(grid,),
        in_specs=[
            pl.BlockSpec((blk, D), lambda i: (i, 0)),
            pl.BlockSpec((blk, D), lambda i: (i, 0)),
            pl.BlockSpec((blk, 4), lambda i: (i, 0)),
            pl.BlockSpec((4, D), lambda i: (0, 0)),
            pl.BlockSpec((1, D), lambda i: (0, 0)),
            pl.BlockSpec((1, D), lambda i: (0, 0)),
            pl.BlockSpec((1, D), lambda i: (0, 0)),
            pl.BlockSpec((D, D), lambda i: (0, 0)),
        ],
        out_specs=pl.BlockSpec((blk, D), lambda i: (i, 0)),
        out_shape=jax.ShapeDtypeStruct((N_EDGES, D), jnp.float32),
    )(pd, qs, ea, w1e, b1, gamma, beta, gmat)


def _scatter_kernel(a_hbm, dst_hbm, zrow_hbm, ones_hbm,
                    sp_hbm, cp_hbm,
                    idx_v, abuf, s_sh):
    c = lax.axis_index("c")
    s = lax.axis_index("s")
    wid = s * NC + c
    base = wid * EDGES_PER_WORKER
    rbase = pl.multiple_of(s * ROWS_PER_SUB, 8)
    rows = s_sh.at[pl.ds(rbase, ROWS_PER_SUB)]

    pltpu.sync_copy(zrow_hbm, rows)
    pltpu.sync_copy(ones_hbm, abuf)
    plsc.subcore_barrier()

    @pl.loop(0, NBLK)
    def _(j):
        off = pl.multiple_of(base + j * EBLK, 8)
        pltpu.sync_copy(dst_hbm.at[pl.ds(off, EBLK)], idx_v)
        pltpu.sync_copy(abuf, s_sh.at[idx_v], add=True)

    plsc.subcore_barrier()
    pltpu.sync_copy(rows, cp_hbm.at[c, pl.ds(rbase, ROWS_PER_SUB)])
    plsc.subcore_barrier()

    pltpu.sync_copy(zrow_hbm, rows)
    plsc.subcore_barrier()

    @pl.loop(0, NBLK)
    def _(j):
        off = pl.multiple_of(base + j * EBLK, 8)
        pltpu.sync_copy(a_hbm.at[pl.ds(off, EBLK)], abuf)
        pltpu.sync_copy(dst_hbm.at[pl.ds(off, EBLK)], idx_v)
        pltpu.sync_copy(abuf, s_sh.at[idx_v], add=True)

    plsc.subcore_barrier()
    pltpu.sync_copy(rows, sp_hbm.at[c, pl.ds(rbase, ROWS_PER_SUB)])


def _scatter_call(a, dst, zrow, ones):
    mesh = plsc.VectorSubcoreMesh(core_axis_name="c", subcore_axis_name="s")
    f = pl.kernel(
        _scatter_kernel,
        mesh=mesh,
        out_type=(
            jax.ShapeDtypeStruct((NC, N_PAD, D), jnp.float32),
            jax.ShapeDtypeStruct((NC, N_PAD, D), jnp.float32),
        ),
        scratch_types=[
            pltpu.VMEM((EBLK,), jnp.int32),
            pltpu.VMEM((EBLK, D), jnp.float32),
            pltpu.VMEM_SHARED((N_PAD, D), jnp.float32),
        ],
    )
    return f(a, dst, zrow, ones)


def _out_body(sp_ref, cp_ref, w2_ref, b2_ref, o_ref):
    sv = sp_ref[0] + sp_ref[1]
    cnt = cp_ref[0, :, 0:1] + cp_ref[1, :, 0:1]
    msg = jnp.dot(sv, w2_ref[...], precision=_HI) + cnt * b2_ref[...]
    o_ref[...] = msg / jnp.maximum(cnt, 1.0)


def _out_call(sp, cp, w2, b2):
    blk = 400
    grid = N_NODES // blk
    return pl.pallas_call(
        _out_body,
        grid=(grid,),
        in_specs=[
            pl.BlockSpec((NC, blk, D), lambda i: (0, i, 0)),
            pl.BlockSpec((NC, blk, D), lambda i: (0, i, 0)),
            pl.BlockSpec((D, D), lambda i: (0, 0)),
            pl.BlockSpec((1, D), lambda i: (0, 0)),
        ],
        out_specs=pl.BlockSpec((blk, D), lambda i: (i, 0)),
        out_shape=jax.ShapeDtypeStruct((N_NODES, D), jnp.float32),
    )(sp, cp, w2, b2)


def kernel(x, edge_index, edge_attr, W1, b1, gamma, beta, W2, b2):
    src = edge_index[0]
    dst = edge_index[1]
    w1d = W1[0:D]
    w1s = W1[D:2 * D]
    w1e = W1[2 * D:]
    gmat = jnp.asarray(_GM_NP)
    zrow = jnp.zeros((ROWS_PER_SUB, D), jnp.float32)
    ones = jnp.ones((EBLK, D), jnp.float32)

    p, q = _pq_call(x, w1d, w1s)
    pd, qs = _gather_call(p, q, dst, src)
    a = _mlp_call(pd, qs, edge_attr, w1e,
                  b1.reshape(1, D), gamma.reshape(1, D), beta.reshape(1, D),
                  gmat)
    sp, cp = _scatter_call(a, dst, zrow, ones)
    return _out_call(sp, cp, W2, b2.reshape(1, D))


# speedup vs baseline: 3.0114x; 1.1064x over previous
import functools

import jax
import jax.numpy as jnp
import numpy as np
from jax import lax
from jax.experimental import pallas as pl
from jax.experimental.pallas import tpu as pltpu
from jax.experimental.pallas import tpu_sc as plsc

N_NODES = 10000
N_EDGES = 320000
D = 128
N_GROUPS = 8
GROUP_SIZE = 16
EPS = 1e-5

NC = 2
NS = 16
NW = NC * NS
EDGES_PER_WORKER = N_EDGES // NW
EBLK = 80
NBLK = EDGES_PER_WORKER // EBLK
N_PAD = 10240
ROWS_PER_SUB = N_PAD // NS

_HI = lax.Precision.HIGHEST

_GM_NP = np.kron(np.eye(N_GROUPS, dtype=np.float32),
                 np.ones((GROUP_SIZE, GROUP_SIZE), dtype=np.float32)) / GROUP_SIZE


def _pq_body(x_ref, wd_ref, ws_ref, p_ref, q_ref):
    xv = x_ref[...]
    p_ref[...] = jnp.dot(xv, wd_ref[...], precision=_HI)
    q_ref[...] = jnp.dot(xv, ws_ref[...], precision=_HI)


def _pq_call(x, wd, ws):
    blk = 400
    grid = N_NODES // blk
    return pl.pallas_call(
        _pq_body,
        grid=(grid,),
        in_specs=[
            pl.BlockSpec((blk, D), lambda i: (i, 0)),
            pl.BlockSpec((D, D), lambda i: (0, 0)),
            pl.BlockSpec((D, D), lambda i: (0, 0)),
        ],
        out_specs=[
            pl.BlockSpec((blk, D), lambda i: (i, 0)),
            pl.BlockSpec((blk, D), lambda i: (i, 0)),
        ],
        out_shape=[
            jax.ShapeDtypeStruct((N_NODES, D), jnp.float32),
            jax.ShapeDtypeStruct((N_NODES, D), jnp.float32),
        ],
    )(x, wd, ws)


EBLK_G = 40
NBLK_G = EDGES_PER_WORKER // EBLK_G
NSLOT = 3
NIT_G = 41


def _gather_kernel(p_hbm, q_hbm, dst3_hbm, src3_hbm, pd_hbm, qs_hbm,
                   dibuf, sibuf,
                   pa0, pa1, pa2, pb0, pb1, pb2,
                   qa0, qa1, qa2, qb0, qb1, qb2,
                   gsem, wsem):
    c = lax.axis_index("c")
    s = lax.axis_index("s")
    wid = s * NC + c
    ebase = wid * EDGES_PER_WORKER
    pA, pB = (pa0, pa1, pa2), (pb0, pb1, pb2)
    qA, qB = (qa0, qa1, qa2), (qb0, qb1, qb2)

    pltpu.sync_copy(dst3_hbm.at[wid], dibuf)
    pltpu.sync_copy(src3_hbm.at[wid], sibuf)

    def drain_writes(bufs):
        for b in bufs:
            pltpu.make_async_copy(b, pd_hbm.at[pl.ds(0, EBLK_G)], wsem).wait()

    def fire_gathers(j0, pset, qset):
        for b in range(NSLOT):
            pltpu.async_copy(p_hbm.at[dibuf.at[j0 + b]], pset[b], gsem)
            pltpu.async_copy(q_hbm.at[sibuf.at[j0 + b]], qset[b], gsem)

    def wait_gathers(pset, qset):
        for b in range(NSLOT):
            pltpu.make_async_copy(p_hbm.at[pl.ds(0, EBLK_G)], pset[b], gsem).wait()
            pltpu.make_async_copy(q_hbm.at[pl.ds(0, EBLK_G)], qset[b], gsem).wait()

    def fire_writes(j0, pset, qset):
        for b in range(NSLOT):
            off = pl.multiple_of(ebase + (j0 + b) * EBLK_G, 8)
            pltpu.async_copy(pset[b], pd_hbm.at[pl.ds(off, EBLK_G)], wsem)
            pltpu.async_copy(qset[b], qs_hbm.at[pl.ds(off, EBLK_G)], wsem)

    @pl.loop(0, NIT_G)
    def _(t):
        j0 = t * 2 * NSLOT

        @pl.when(t > 0)
        def _():
            drain_writes(pA)
            drain_writes(qA)
        fire_gathers(j0, pA, qA)

        @pl.when(t > 0)
        def _():
            drain_writes(pB)
            drain_writes(qB)
        fire_gathers(j0 + NSLOT, pB, qB)

        wait_gathers(pA, qA)
        fire_writes(j0, pA, qA)
        wait_gathers(pB, qB)
        fire_writes(j0 + NSLOT, pB, qB)

    drain_writes(pA)
    drain_writes(qA)
    drain_writes(pB)
    drain_writes(qB)

    for j, pbuf, qbuf in ((NIT_G * 2 * NSLOT, pA[0], qA[0]),
                          (NIT_G * 2 * NSLOT + 1, pA[1], qA[1]),
                          (NIT_G * 2 * NSLOT + 2, pA[2], qA[2]),
                          (NIT_G * 2 * NSLOT + 3, pB[0], qB[0])):
        off = pl.multiple_of(ebase + j * EBLK_G, 8)
        cp1 = pltpu.async_copy(p_hbm.at[dibuf.at[j]], pbuf, gsem)
        cp2 = pltpu.async_copy(q_hbm.at[sibuf.at[j]], qbuf, gsem)
        cp1.wait()
        cp2.wait()
        pltpu.sync_copy(pbuf, pd_hbm.at[pl.ds(off, EBLK_G)])
        pltpu.sync_copy(qbuf, qs_hbm.at[pl.ds(off, EBLK_G)])


def _gather_call(p, q, dst3, src3):
    mesh = plsc.VectorSubcoreMesh(core_axis_name="c", subcore_axis_name="s")
    rowbufs = [pltpu.VMEM((EBLK_G, D), jnp.float32)] * (4 * NSLOT)
    f = pl.kernel(
        _gather_kernel,
        mesh=mesh,
        out_type=(
            jax.ShapeDtypeStruct((N_EDGES, D), jnp.float32),
            jax.ShapeDtypeStruct((N_EDGES, D), jnp.float32),
        ),
        scratch_types=[
            pltpu.VMEM((NBLK_G, EBLK_G), jnp.int32),
            pltpu.VMEM((NBLK_G, EBLK_G), jnp.int32),
            *rowbufs,
            pltpu.SemaphoreType.DMA,
            pltpu.SemaphoreType.DMA,
        ],
    )
    return f(p, q, dst3, src3)


def _mlp_body(pd_ref, qs_ref, ea_ref, w1e_ref, b1_ref, g_ref, bt_ref, gm_ref,
              a_ref):
    h = (pd_ref[...] + qs_ref[...]
         + jnp.dot(ea_ref[...], w1e_ref[...], precision=_HI) + b1_ref[...])
    gm = gm_ref[...]
    m = jnp.dot(h, gm, precision=_HI)
    sq = jnp.dot(h * h, gm, precision=_HI)
    var = sq - m * m
    y = (h - m) * lax.rsqrt(var + EPS) * g_ref[...] + bt_ref[...]
    a_ref[...] = y * jax.nn.sigmoid(y)


def _mlp_call(pd, qs, ea, w1e, b1, gamma, beta, gmat):
    blk = 1600
    grid = N_EDGES // blk
    return pl.pallas_call(
        _mlp_body,
        grid=(grid,),
        in_specs=[
            pl.BlockSpec((blk, D), lambda i: (i, 0)),
            pl.BlockSpec((blk, D), lambda i: (i, 0)),
            pl.BlockSpec((blk, 4), lambda i: (i, 0)),
            pl.BlockSpec((4, D), lambda i: (0, 0)),
            pl.BlockSpec((1, D), lambda i: (0, 0)),
            pl.BlockSpec((1, D), lambda i: (0, 0)),
            pl.BlockSpec((1, D), lambda i: (0, 0)),
            pl.BlockSpec((D, D), lambda i: (0, 0)),
        ],
        out_specs=pl.BlockSpec((blk, D), lambda i: (i, 0)),
        out_shape=jax.ShapeDtypeStruct((N_EDGES, D), jnp.float32),
    )(pd, qs, ea, w1e, b1, gamma, beta, gmat)


def _scatter_kernel(a_hbm, dst_hbm, zrow_hbm, ones_hbm,
                    sp_hbm, cp_hbm,
                    idx_v, abuf, s_sh):
    c = lax.axis_index("c")
    s = lax.axis_index("s")
    wid = s * NC + c
    base = wid * EDGES_PER_WORKER
    rbase = pl.multiple_of(s * ROWS_PER_SUB, 8)
    rows = s_sh.at[pl.ds(rbase, ROWS_PER_SUB)]

    pltpu.sync_copy(zrow_hbm, rows)
    pltpu.sync_copy(ones_hbm, abuf)
    plsc.subcore_barrier()

    @pl.loop(0, NBLK)
    def _(j):
        off = pl.multiple_of(base + j * EBLK, 8)
        pltpu.sync_copy(dst_hbm.at[pl.ds(off, EBLK)], idx_v)
        pltpu.sync_copy(abuf, s_sh.at[idx_v], add=True)

    plsc.subcore_barrier()
    pltpu.sync_copy(rows, cp_hbm.at[c, pl.ds(rbase, ROWS_PER_SUB)])
    plsc.subcore_barrier()

    pltpu.sync_copy(zrow_hbm, rows)
    plsc.subcore_barrier()

    @pl.loop(0, NBLK)
    def _(j):
        off = pl.multiple_of(base + j * EBLK, 8)
        pltpu.sync_copy(a_hbm.at[pl.ds(off, EBLK)], abuf)
        pltpu.sync_copy(dst_hbm.at[pl.ds(off, EBLK)], idx_v)
        pltpu.sync_copy(abuf, s_sh.at[idx_v], add=True)

    plsc.subcore_barrier()
    pltpu.sync_copy(rows, sp_hbm.at[c, pl.ds(rbase, ROWS_PER_SUB)])


def _scatter_call(a, dst, zrow, ones):
    mesh = plsc.VectorSubcoreMesh(core_axis_name="c", subcore_axis_name="s")
    f = pl.kernel(
        _scatter_kernel,
        mesh=mesh,
        out_type=(
            jax.ShapeDtypeStruct((NC, N_PAD, D), jnp.float32),
            jax.ShapeDtypeStruct((NC, N_PAD, D), jnp.float32),
        ),
        scratch_types=[
            pltpu.VMEM((EBLK,), jnp.int32),
            pltpu.VMEM((EBLK, D), jnp.float32),
            pltpu.VMEM_SHARED((N_PAD, D), jnp.float32),
        ],
    )
    return f(a, dst, zrow, ones)


def _out_body(sp_ref, cp_ref, w2_ref, b2_ref, o_ref):
    sv = sp_ref[0] + sp_ref[1]
    cnt = cp_ref[0, :, 0:1] + cp_ref[1, :, 0:1]
    msg = jnp.dot(sv, w2_ref[...], precision=_HI) + cnt * b2_ref[...]
    o_ref[...] = msg / jnp.maximum(cnt, 1.0)


def _out_call(sp, cp, w2, b2):
    blk = 400
    grid = N_NODES // blk
    return pl.pallas_call(
        _out_body,
        grid=(grid,),
        in_specs=[
            pl.BlockSpec((NC, blk, D), lambda i: (0, i, 0)),
            pl.BlockSpec((NC, blk, D), lambda i: (0, i, 0)),
            pl.BlockSpec((D, D), lambda i: (0, 0)),
            pl.BlockSpec((1, D), lambda i: (0, 0)),
        ],
        out_specs=pl.BlockSpec((blk, D), lambda i: (i, 0)),
        out_shape=jax.ShapeDtypeStruct((N_NODES, D), jnp.float32),
    )(sp, cp, w2, b2)


def kernel(x, edge_index, edge_attr, W1, b1, gamma, beta, W2, b2):
    src = edge_index[0]
    dst = edge_index[1]
    w1d = W1[0:D]
    w1s = W1[D:2 * D]
    w1e = W1[2 * D:]
    gmat = jnp.asarray(_GM_NP)
    zrow = jnp.zeros((ROWS_PER_SUB, D), jnp.float32)
    ones = jnp.ones((EBLK, D), jnp.float32)

    p, q = _pq_call(x, w1d, w1s)
    dst3 = dst.reshape(NW, NBLK_G, EBLK_G)
    src3 = src.reshape(NW, NBLK_G, EBLK_G)
    pd, qs = _gather_call(p, q, dst3, src3)
    a = _mlp_call(pd, qs, edge_attr, w1e,
                  b1.reshape(1, D), gamma.reshape(1, D), beta.reshape(1, D),
                  gmat)
    sp, cp = _scatter_call(a, dst, zrow, ones)
    return _out_call(sp, cp, W2, b2.reshape(1, D))


# speedup vs baseline: 3.4435x; 1.1435x over previous
import functools

import jax
import jax.numpy as jnp
import numpy as np
from jax import lax
from jax.experimental import pallas as pl
from jax.experimental.pallas import tpu as pltpu
from jax.experimental.pallas import tpu_sc as plsc

N_NODES = 10000
N_EDGES = 320000
D = 128
N_GROUPS = 8
GROUP_SIZE = 16
EPS = 1e-5

NC = 2
NS = 16
NW = NC * NS
EDGES_PER_WORKER = N_EDGES // NW
EBLK = 80
NBLK = EDGES_PER_WORKER // EBLK
N_PAD = 10240
ROWS_PER_SUB = N_PAD // NS

_HI = lax.Precision.HIGHEST

_GM_NP = np.kron(np.eye(N_GROUPS, dtype=np.float32),
                 np.ones((GROUP_SIZE, GROUP_SIZE), dtype=np.float32)) / GROUP_SIZE


def _pq_body(x_ref, wd_ref, ws_ref, p_ref, q_ref):
    xv = x_ref[...]
    p_ref[...] = jnp.dot(xv, wd_ref[...], precision=_HI)
    q_ref[...] = jnp.dot(xv, ws_ref[...], precision=_HI)


def _pq_call(x, wd, ws):
    blk = 400
    grid = N_NODES // blk
    return pl.pallas_call(
        _pq_body,
        grid=(grid,),
        in_specs=[
            pl.BlockSpec((blk, D), lambda i: (i, 0)),
            pl.BlockSpec((D, D), lambda i: (0, 0)),
            pl.BlockSpec((D, D), lambda i: (0, 0)),
        ],
        out_specs=[
            pl.BlockSpec((blk, D), lambda i: (i, 0)),
            pl.BlockSpec((blk, D), lambda i: (i, 0)),
        ],
        out_shape=[
            jax.ShapeDtypeStruct((N_NODES, D), jnp.float32),
            jax.ShapeDtypeStruct((N_NODES, D), jnp.float32),
        ],
    )(x, wd, ws)


EBLK_G = 40
NBLK_G = EDGES_PER_WORKER // EBLK_G
NSLOT = 3
NIT_G = 41


def _gather_kernel(p_hbm, q_hbm, dst3_hbm, src3_hbm, pd_hbm, qs_hbm,
                   dibuf, sibuf,
                   pa0, pa1, pa2, pb0, pb1, pb2,
                   qa0, qa1, qa2, qb0, qb1, qb2,
                   gsem, wsem):
    c = lax.axis_index("c")
    s = lax.axis_index("s")
    wid = s * NC + c
    ebase = wid * EDGES_PER_WORKER
    pA, pB = (pa0, pa1, pa2), (pb0, pb1, pb2)
    qA, qB = (qa0, qa1, qa2), (qb0, qb1, qb2)

    pltpu.sync_copy(dst3_hbm.at[wid], dibuf)
    pltpu.sync_copy(src3_hbm.at[wid], sibuf)

    def drain_writes(bufs):
        for b in bufs:
            pltpu.make_async_copy(b, pd_hbm.at[pl.ds(0, EBLK_G)], wsem).wait()

    def fire_gathers(j0, pset, qset):
        for b in range(NSLOT):
            pltpu.async_copy(p_hbm.at[dibuf.at[j0 + b]], pset[b], gsem)
            pltpu.async_copy(q_hbm.at[sibuf.at[j0 + b]], qset[b], gsem)

    def wait_gathers(pset, qset):
        for b in range(NSLOT):
            pltpu.make_async_copy(p_hbm.at[pl.ds(0, EBLK_G)], pset[b], gsem).wait()
            pltpu.make_async_copy(q_hbm.at[pl.ds(0, EBLK_G)], qset[b], gsem).wait()

    def fire_writes(j0, pset, qset):
        for b in range(NSLOT):
            off = pl.multiple_of(ebase + (j0 + b) * EBLK_G, 8)
            pltpu.async_copy(pset[b], pd_hbm.at[pl.ds(off, EBLK_G)], wsem)
            pltpu.async_copy(qset[b], qs_hbm.at[pl.ds(off, EBLK_G)], wsem)

    @pl.loop(0, NIT_G)
    def _(t):
        j0 = t * 2 * NSLOT

        @pl.when(t > 0)
        def _():
            drain_writes(pA)
            drain_writes(qA)
        fire_gathers(j0, pA, qA)

        @pl.when(t > 0)
        def _():
            drain_writes(pB)
            drain_writes(qB)
        fire_gathers(j0 + NSLOT, pB, qB)

        wait_gathers(pA, qA)
        fire_writes(j0, pA, qA)
        wait_gathers(pB, qB)
        fire_writes(j0 + NSLOT, pB, qB)

    drain_writes(pA)
    drain_writes(qA)
    drain_writes(pB)
    drain_writes(qB)

    for j, pbuf, qbuf in ((NIT_G * 2 * NSLOT, pA[0], qA[0]),
                          (NIT_G * 2 * NSLOT + 1, pA[1], qA[1]),
                          (NIT_G * 2 * NSLOT + 2, pA[2], qA[2]),
                          (NIT_G * 2 * NSLOT + 3, pB[0], qB[0])):
        off = pl.multiple_of(ebase + j * EBLK_G, 8)
        cp1 = pltpu.async_copy(p_hbm.at[dibuf.at[j]], pbuf, gsem)
        cp2 = pltpu.async_copy(q_hbm.at[sibuf.at[j]], qbuf, gsem)
        cp1.wait()
        cp2.wait()
        pltpu.sync_copy(pbuf, pd_hbm.at[pl.ds(off, EBLK_G)])
        pltpu.sync_copy(qbuf, qs_hbm.at[pl.ds(off, EBLK_G)])


def _gather_call(p, q, dst3, src3):
    mesh = plsc.VectorSubcoreMesh(core_axis_name="c", subcore_axis_name="s")
    rowbufs = [pltpu.VMEM((EBLK_G, D), jnp.float32)] * (4 * NSLOT)
    f = pl.kernel(
        _gather_kernel,
        mesh=mesh,
        out_type=(
            jax.ShapeDtypeStruct((N_EDGES, D), jnp.float32),
            jax.ShapeDtypeStruct((N_EDGES, D), jnp.float32),
        ),
        scratch_types=[
            pltpu.VMEM((NBLK_G, EBLK_G), jnp.int32),
            pltpu.VMEM((NBLK_G, EBLK_G), jnp.int32),
            *rowbufs,
            pltpu.SemaphoreType.DMA,
            pltpu.SemaphoreType.DMA,
        ],
    )
    return f(p, q, dst3, src3)


def _mlp_body(pd_ref, qs_ref, ea_ref, w1e_ref, b1_ref, g_ref, bt_ref, gm_ref,
              a_ref):
    h = (pd_ref[...] + qs_ref[...]
         + jnp.dot(ea_ref[...], w1e_ref[...], precision=_HI) + b1_ref[...])
    gm = gm_ref[...]
    m = jnp.dot(h, gm, precision=_HI)
    sq = jnp.dot(h * h, gm, precision=_HI)
    var = sq - m * m
    y = (h - m) * lax.rsqrt(var + EPS) * g_ref[...] + bt_ref[...]
    a_ref[...] = y * jax.nn.sigmoid(y)


def _mlp_call(pd, qs, ea, w1e, b1, gamma, beta, gmat):
    blk = 1600
    grid = N_EDGES // blk
    return pl.pallas_call(
        _mlp_body,
        grid=(grid,),
        in_specs=[
            pl.BlockSpec((blk, D), lambda i: (i, 0)),
            pl.BlockSpec((blk, D), lambda i: (i, 0)),
            pl.BlockSpec((blk, 4), lambda i: (i, 0)),
            pl.BlockSpec((4, D), lambda i: (0, 0)),
            pl.BlockSpec((1, D), lambda i: (0, 0)),
            pl.BlockSpec((1, D), lambda i: (0, 0)),
            pl.BlockSpec((1, D), lambda i: (0, 0)),
            pl.BlockSpec((D, D), lambda i: (0, 0)),
        ],
        out_specs=pl.BlockSpec((blk, D), lambda i: (i, 0)),
        out_shape=jax.ShapeDtypeStruct((N_EDGES, D), jnp.float32),
    )(pd, qs, ea, w1e, b1, gamma, beta, gmat)


def _scatter_kernel(a_hbm, dst3_hbm, zrow_hbm, ones_hbm,
                    sp_hbm, cp_hbm,
                    idxbuf, abuf0, abuf1, onesbuf, s_sh, csem, asem0, asem1):
    c = lax.axis_index("c")
    s = lax.axis_index("s")
    wid = s * NC + c
    ebase = wid * EDGES_PER_WORKER
    rbase = pl.multiple_of(s * ROWS_PER_SUB, 8)
    rows = s_sh.at[pl.ds(rbase, ROWS_PER_SUB)]

    pltpu.sync_copy(dst3_hbm.at[wid], idxbuf)
    pltpu.sync_copy(ones_hbm, onesbuf)

    pltpu.sync_copy(zrow_hbm, rows)
    plsc.subcore_barrier()

    @pl.loop(0, NBLK)
    def _(j):
        pltpu.async_copy(onesbuf, s_sh.at[idxbuf.at[j]], csem, add=True)

    @pl.loop(0, NBLK)
    def _(j):
        pltpu.make_async_copy(onesbuf, s_sh.at[idxbuf.at[0]], csem).wait()

    plsc.subcore_barrier()
    pltpu.sync_copy(rows, cp_hbm.at[c, pl.ds(rbase, ROWS_PER_SUB)])
    plsc.subcore_barrier()

    pltpu.sync_copy(zrow_hbm, rows)
    plsc.subcore_barrier()

    off0 = pl.multiple_of(ebase, 8)
    pltpu.sync_copy(a_hbm.at[pl.ds(off0, EBLK)], abuf0)

    @pl.loop(0, (NBLK - 1) // 2)
    def _(t):
        j0 = 2 * t
        add0 = pltpu.async_copy(abuf0, s_sh.at[idxbuf.at[j0]], asem0, add=True)
        off1 = pl.multiple_of(ebase + (j0 + 1) * EBLK, 8)
        pltpu.sync_copy(a_hbm.at[pl.ds(off1, EBLK)], abuf1)
        add0.wait()
        add1 = pltpu.async_copy(abuf1, s_sh.at[idxbuf.at[j0 + 1]], asem1, add=True)
        off2 = pl.multiple_of(ebase + (j0 + 2) * EBLK, 8)
        pltpu.sync_copy(a_hbm.at[pl.ds(off2, EBLK)], abuf0)
        add1.wait()

    pltpu.sync_copy(abuf0, s_sh.at[idxbuf.at[NBLK - 1]], add=True)

    plsc.subcore_barrier()
    pltpu.sync_copy(rows, sp_hbm.at[c, pl.ds(rbase, ROWS_PER_SUB)])


def _scatter_call(a, dst3, zrow, ones):
    mesh = plsc.VectorSubcoreMesh(core_axis_name="c", subcore_axis_name="s")
    f = pl.kernel(
        _scatter_kernel,
        mesh=mesh,
        out_type=(
            jax.ShapeDtypeStruct((NC, N_PAD, D), jnp.float32),
            jax.ShapeDtypeStruct((NC, N_PAD, D), jnp.float32),
        ),
        scratch_types=[
            pltpu.VMEM((NBLK, EBLK), jnp.int32),
            pltpu.VMEM((EBLK, D), jnp.float32),
            pltpu.VMEM((EBLK, D), jnp.float32),
            pltpu.VMEM((EBLK, D), jnp.float32),
            pltpu.VMEM_SHARED((N_PAD, D), jnp.float32),
            pltpu.SemaphoreType.DMA,
            pltpu.SemaphoreType.DMA,
            pltpu.SemaphoreType.DMA,
        ],
    )
    return f(a, dst3, zrow, ones)


def _out_body(sp_ref, cp_ref, w2_ref, b2_ref, o_ref):
    sv = sp_ref[0] + sp_ref[1]
    cnt = cp_ref[0, :, 0:1] + cp_ref[1, :, 0:1]
    msg = jnp.dot(sv, w2_ref[...], precision=_HI) + cnt * b2_ref[...]
    o_ref[...] = msg / jnp.maximum(cnt, 1.0)


def _out_call(sp, cp, w2, b2):
    blk = 400
    grid = N_NODES // blk
    return pl.pallas_call(
        _out_body,
        grid=(grid,),
        in_specs=[
            pl.BlockSpec((NC, blk, D), lambda i: (0, i, 0)),
            pl.BlockSpec((NC, blk, D), lambda i: (0, i, 0)),
            pl.BlockSpec((D, D), lambda i: (0, 0)),
            pl.BlockSpec((1, D), lambda i: (0, 0)),
        ],
        out_specs=pl.BlockSpec((blk, D), lambda i: (i, 0)),
        out_shape=jax.ShapeDtypeStruct((N_NODES, D), jnp.float32),
    )(sp, cp, w2, b2)


def kernel(x, edge_index, edge_attr, W1, b1, gamma, beta, W2, b2):
    src = edge_index[0]
    dst = edge_index[1]
    w1d = W1[0:D]
    w1s = W1[D:2 * D]
    w1e = W1[2 * D:]
    gmat = jnp.asarray(_GM_NP)
    zrow = jnp.zeros((ROWS_PER_SUB, D), jnp.float32)
    ones = jnp.ones((EBLK, D), jnp.float32)

    p, q = _pq_call(x, w1d, w1s)
    dst3 = dst.reshape(NW, NBLK_G, EBLK_G)
    src3 = src.reshape(NW, NBLK_G, EBLK_G)
    pd, qs = _gather_call(p, q, dst3, src3)
    a = _mlp_call(pd, qs, edge_attr, w1e,
                  b1.reshape(1, D), gamma.reshape(1, D), beta.reshape(1, D),
                  gmat)
    dstS = dst.reshape(NW, NBLK, EBLK)
    sp, cp = _scatter_call(a, dstS, zrow, ones)
    return _out_call(sp, cp, W2, b2.reshape(1, D))


# speedup vs baseline: 4.1383x; 1.2018x over previous
import functools

import jax
import jax.numpy as jnp
import numpy as np
from jax import lax
from jax.experimental import pallas as pl
from jax.experimental.pallas import tpu as pltpu
from jax.experimental.pallas import tpu_sc as plsc

N_NODES = 10000
N_EDGES = 320000
D = 128
N_GROUPS = 8
GROUP_SIZE = 16
EPS = 1e-5

NC = 2
NS = 16
NW = NC * NS
EDGES_PER_WORKER = N_EDGES // NW
EBLK = 80
NBLK = EDGES_PER_WORKER // EBLK
N_PAD = 10240
ROWS_PER_SUB = N_PAD // NS

_HI = lax.Precision.HIGHEST

_GM_NP = np.kron(np.eye(N_GROUPS, dtype=np.float32),
                 np.ones((GROUP_SIZE, GROUP_SIZE), dtype=np.float32)) / GROUP_SIZE


def _pq_body(x_ref, wd_ref, ws_ref, p_ref, q_ref):
    xv = x_ref[...]
    p_ref[...] = jnp.dot(xv, wd_ref[...], precision=_HI)
    q_ref[...] = jnp.dot(xv, ws_ref[...], precision=_HI)


def _pq_call(x, wd, ws):
    blk = 400
    grid = N_NODES // blk
    return pl.pallas_call(
        _pq_body,
        grid=(grid,),
        in_specs=[
            pl.BlockSpec((blk, D), lambda i: (i, 0)),
            pl.BlockSpec((D, D), lambda i: (0, 0)),
            pl.BlockSpec((D, D), lambda i: (0, 0)),
        ],
        out_specs=[
            pl.BlockSpec((blk, D), lambda i: (i, 0)),
            pl.BlockSpec((blk, D), lambda i: (i, 0)),
        ],
        out_shape=[
            jax.ShapeDtypeStruct((N_NODES, D), jnp.float32),
            jax.ShapeDtypeStruct((N_NODES, D), jnp.float32),
        ],
    )(x, wd, ws)


EBLK_G = 40
NBLK_G = EDGES_PER_WORKER // EBLK_G
NSLOT = 3
NIT_G = 41


def _gather_kernel(p_hbm, q_hbm, dst3_hbm, src3_hbm, pd_hbm, qs_hbm,
                   dibuf, sibuf,
                   pa0, pa1, pa2, pb0, pb1, pb2,
                   qa0, qa1, qa2, qb0, qb1, qb2,
                   gsem, wsem):
    c = lax.axis_index("c")
    s = lax.axis_index("s")
    wid = s * NC + c
    ebase = wid * EDGES_PER_WORKER
    pA, pB = (pa0, pa1, pa2), (pb0, pb1, pb2)
    qA, qB = (qa0, qa1, qa2), (qb0, qb1, qb2)

    pltpu.sync_copy(dst3_hbm.at[wid], dibuf)
    pltpu.sync_copy(src3_hbm.at[wid], sibuf)

    def drain_writes(bufs):
        for b in bufs:
            pltpu.make_async_copy(b, pd_hbm.at[pl.ds(0, EBLK_G)], wsem).wait()

    def fire_gathers(j0, pset, qset):
        for b in range(NSLOT):
            pltpu.async_copy(p_hbm.at[dibuf.at[j0 + b]], pset[b], gsem)
            pltpu.async_copy(q_hbm.at[sibuf.at[j0 + b]], qset[b], gsem)

    def wait_gathers(pset, qset):
        for b in range(NSLOT):
            pltpu.make_async_copy(p_hbm.at[pl.ds(0, EBLK_G)], pset[b], gsem).wait()
            pltpu.make_async_copy(q_hbm.at[pl.ds(0, EBLK_G)], qset[b], gsem).wait()

    def fire_writes(j0, pset, qset):
        for b in range(NSLOT):
            off = pl.multiple_of(ebase + (j0 + b) * EBLK_G, 8)
            pltpu.async_copy(pset[b], pd_hbm.at[pl.ds(off, EBLK_G)], wsem)
            pltpu.async_copy(qset[b], qs_hbm.at[pl.ds(off, EBLK_G)], wsem)

    @pl.loop(0, NIT_G)
    def _(t):
        j0 = t * 2 * NSLOT

        @pl.when(t > 0)
        def _():
            drain_writes(pA)
            drain_writes(qA)
        fire_gathers(j0, pA, qA)

        @pl.when(t > 0)
        def _():
            drain_writes(pB)
            drain_writes(qB)
        fire_gathers(j0 + NSLOT, pB, qB)

        wait_gathers(pA, qA)
        fire_writes(j0, pA, qA)
        wait_gathers(pB, qB)
        fire_writes(j0 + NSLOT, pB, qB)

    drain_writes(pA)
    drain_writes(qA)
    drain_writes(pB)
    drain_writes(qB)

    for j, pbuf, qbuf in ((NIT_G * 2 * NSLOT, pA[0], qA[0]),
                          (NIT_G * 2 * NSLOT + 1, pA[1], qA[1]),
                          (NIT_G * 2 * NSLOT + 2, pA[2], qA[2]),
                          (NIT_G * 2 * NSLOT + 3, pB[0], qB[0])):
        off = pl.multiple_of(ebase + j * EBLK_G, 8)
        cp1 = pltpu.async_copy(p_hbm.at[dibuf.at[j]], pbuf, gsem)
        cp2 = pltpu.async_copy(q_hbm.at[sibuf.at[j]], qbuf, gsem)
        cp1.wait()
        cp2.wait()
        pltpu.sync_copy(pbuf, pd_hbm.at[pl.ds(off, EBLK_G)])
        pltpu.sync_copy(qbuf, qs_hbm.at[pl.ds(off, EBLK_G)])


def _gather_call(p, q, dst3, src3):
    mesh = plsc.VectorSubcoreMesh(core_axis_name="c", subcore_axis_name="s")
    rowbufs = [pltpu.VMEM((EBLK_G, D), jnp.float32)] * (4 * NSLOT)
    f = pl.kernel(
        _gather_kernel,
        mesh=mesh,
        out_type=(
            jax.ShapeDtypeStruct((N_EDGES, D), jnp.float32),
            jax.ShapeDtypeStruct((N_EDGES, D), jnp.float32),
        ),
        scratch_types=[
            pltpu.VMEM((NBLK_G, EBLK_G), jnp.int32),
            pltpu.VMEM((NBLK_G, EBLK_G), jnp.int32),
            *rowbufs,
            pltpu.SemaphoreType.DMA,
            pltpu.SemaphoreType.DMA,
        ],
    )
    return f(p, q, dst3, src3)


def _mlp_body(pd_ref, qs_ref, ea_ref, w1e_ref, b1_ref, g_ref, bt_ref, gm_ref,
              a_ref):
    h = (pd_ref[...] + qs_ref[...]
         + jnp.dot(ea_ref[...], w1e_ref[...], precision=_HI) + b1_ref[...])
    gm = gm_ref[...]
    m = jnp.dot(h, gm)
    sq = jnp.dot(h * h, gm)
    var = sq - m * m
    y = (h - m) * lax.rsqrt(var + EPS) * g_ref[...] + bt_ref[...]
    a_ref[...] = y * jax.nn.sigmoid(y)


def _mlp_call(pd, qs, ea, w1e, b1, gamma, beta, gmat):
    blk = 3200
    grid = N_EDGES // blk
    return pl.pallas_call(
        _mlp_body,
        grid=(grid,),
        in_specs=[
            pl.BlockSpec((blk, D), lambda i: (i, 0)),
            pl.BlockSpec((blk, D), lambda i: (i, 0)),
            pl.BlockSpec((blk, 4), lambda i: (i, 0)),
            pl.BlockSpec((4, D), lambda i: (0, 0)),
            pl.BlockSpec((1, D), lambda i: (0, 0)),
            pl.BlockSpec((1, D), lambda i: (0, 0)),
            pl.BlockSpec((1, D), lambda i: (0, 0)),
            pl.BlockSpec((D, D), lambda i: (0, 0)),
        ],
        out_specs=pl.BlockSpec((blk, D), lambda i: (i, 0)),
        out_shape=jax.ShapeDtypeStruct((N_EDGES, D), jnp.float32),
    )(pd, qs, ea, w1e, b1, gamma, beta, gmat)


def _scatter_kernel(a_hbm, dst3_hbm, zrow_hbm, ones_hbm,
                    sp_hbm, cp_hbm,
                    idxbuf, abuf0, abuf1, onesbuf, s_sh, csem, asem0, asem1):
    c = lax.axis_index("c")
    s = lax.axis_index("s")
    wid = s * NC + c
    ebase = wid * EDGES_PER_WORKER
    rbase = pl.multiple_of(s * ROWS_PER_SUB, 8)
    rows = s_sh.at[pl.ds(rbase, ROWS_PER_SUB)]

    pltpu.sync_copy(dst3_hbm.at[wid], idxbuf)
    pltpu.sync_copy(ones_hbm, onesbuf)

    pltpu.sync_copy(zrow_hbm, rows)
    plsc.subcore_barrier()

    @pl.loop(0, NBLK)
    def _(j):
        pltpu.async_copy(onesbuf, s_sh.at[idxbuf.at[j]], csem, add=True)

    @pl.loop(0, NBLK)
    def _(j):
        pltpu.make_async_copy(onesbuf, s_sh.at[idxbuf.at[0]], csem).wait()

    plsc.subcore_barrier()
    pltpu.sync_copy(rows, cp_hbm.at[c, pl.ds(rbase, ROWS_PER_SUB)])
    plsc.subcore_barrier()

    pltpu.sync_copy(zrow_hbm, rows)
    plsc.subcore_barrier()

    off0 = pl.multiple_of(ebase, 8)
    pltpu.sync_copy(a_hbm.at[pl.ds(off0, EBLK)], abuf0)

    @pl.loop(0, (NBLK - 1) // 2)
    def _(t):
        j0 = 2 * t
        add0 = pltpu.async_copy(abuf0, s_sh.at[idxbuf.at[j0]], asem0, add=True)
        off1 = pl.multiple_of(ebase + (j0 + 1) * EBLK, 8)
        pltpu.sync_copy(a_hbm.at[pl.ds(off1, EBLK)], abuf1)
        add0.wait()
        add1 = pltpu.async_copy(abuf1, s_sh.at[idxbuf.at[j0 + 1]], asem1, add=True)
        off2 = pl.multiple_of(ebase + (j0 + 2) * EBLK, 8)
        pltpu.sync_copy(a_hbm.at[pl.ds(off2, EBLK)], abuf0)
        add1.wait()

    pltpu.sync_copy(abuf0, s_sh.at[idxbuf.at[NBLK - 1]], add=True)

    plsc.subcore_barrier()
    pltpu.sync_copy(rows, sp_hbm.at[c, pl.ds(rbase, ROWS_PER_SUB)])


def _scatter_call(a, dst3, zrow, ones):
    mesh = plsc.VectorSubcoreMesh(core_axis_name="c", subcore_axis_name="s")
    f = pl.kernel(
        _scatter_kernel,
        mesh=mesh,
        out_type=(
            jax.ShapeDtypeStruct((NC, N_PAD, D), jnp.float32),
            jax.ShapeDtypeStruct((NC, N_PAD, D), jnp.float32),
        ),
        scratch_types=[
            pltpu.VMEM((NBLK, EBLK), jnp.int32),
            pltpu.VMEM((EBLK, D), jnp.float32),
            pltpu.VMEM((EBLK, D), jnp.float32),
            pltpu.VMEM((EBLK, D), jnp.float32),
            pltpu.VMEM_SHARED((N_PAD, D), jnp.float32),
            pltpu.SemaphoreType.DMA,
            pltpu.SemaphoreType.DMA,
            pltpu.SemaphoreType.DMA,
        ],
    )
    return f(a, dst3, zrow, ones)


def _out_body(sp_ref, cp_ref, w2_ref, b2_ref, o_ref):
    sv = sp_ref[0] + sp_ref[1]
    cnt = cp_ref[0, :, 0:1] + cp_ref[1, :, 0:1]
    msg = jnp.dot(sv, w2_ref[...], precision=_HI) + cnt * b2_ref[...]
    o_ref[...] = msg / jnp.maximum(cnt, 1.0)


def _out_call(sp, cp, w2, b2):
    blk = 400
    grid = N_NODES // blk
    return pl.pallas_call(
        _out_body,
        grid=(grid,),
        in_specs=[
            pl.BlockSpec((NC, blk, D), lambda i: (0, i, 0)),
            pl.BlockSpec((NC, blk, D), lambda i: (0, i, 0)),
            pl.BlockSpec((D, D), lambda i: (0, 0)),
            pl.BlockSpec((1, D), lambda i: (0, 0)),
        ],
        out_specs=pl.BlockSpec((blk, D), lambda i: (i, 0)),
        out_shape=jax.ShapeDtypeStruct((N_NODES, D), jnp.float32),
    )(sp, cp, w2, b2)


def kernel(x, edge_index, edge_attr, W1, b1, gamma, beta, W2, b2):
    src = edge_index[0]
    dst = edge_index[1]
    w1d = W1[0:D]
    w1s = W1[D:2 * D]
    w1e = W1[2 * D:]
    gmat = jnp.asarray(_GM_NP)
    zrow = jnp.zeros((ROWS_PER_SUB, D), jnp.float32)
    ones = jnp.ones((EBLK, D), jnp.float32)

    p, q = _pq_call(x, w1d, w1s)
    dst3 = dst.reshape(NW, NBLK_G, EBLK_G)
    src3 = src.reshape(NW, NBLK_G, EBLK_G)
    pd, qs = _gather_call(p, q, dst3, src3)
    a = _mlp_call(pd, qs, edge_attr, w1e,
                  b1.reshape(1, D), gamma.reshape(1, D), beta.reshape(1, D),
                  gmat)
    dstS = dst.reshape(NW, NBLK, EBLK)
    sp, cp = _scatter_call(a, dstS, zrow, ones)
    return _out_call(sp, cp, W2, b2.reshape(1, D))


# speedup vs baseline: 4.4673x; 1.0795x over previous
import functools

import jax
import jax.numpy as jnp
import numpy as np
from jax import lax
from jax.experimental import pallas as pl
from jax.experimental.pallas import tpu as pltpu
from jax.experimental.pallas import tpu_sc as plsc

N_NODES = 10000
N_EDGES = 320000
D = 128
N_GROUPS = 8
GROUP_SIZE = 16
EPS = 1e-5

NC = 2
NS = 16
NW = NC * NS
EDGES_PER_WORKER = N_EDGES // NW
EBLK = 80
NBLK = EDGES_PER_WORKER // EBLK
N_PAD = 10240
ROWS_PER_SUB = N_PAD // NS

_HI = lax.Precision.HIGHEST

_GM_NP = np.kron(np.eye(N_GROUPS, dtype=np.float32),
                 np.ones((GROUP_SIZE, GROUP_SIZE), dtype=np.float32)) / GROUP_SIZE


def _pq_body(x_ref, wd_ref, ws_ref, p_ref, q_ref):
    xv = x_ref[...]
    p_ref[...] = jnp.dot(xv, wd_ref[...], precision=_HI)
    q_ref[...] = jnp.dot(xv, ws_ref[...], precision=_HI)


def _pq_call(x, wd, ws):
    blk = 400
    grid = N_NODES // blk
    return pl.pallas_call(
        _pq_body,
        grid=(grid,),
        in_specs=[
            pl.BlockSpec((blk, D), lambda i: (i, 0)),
            pl.BlockSpec((D, D), lambda i: (0, 0)),
            pl.BlockSpec((D, D), lambda i: (0, 0)),
        ],
        out_specs=[
            pl.BlockSpec((blk, D), lambda i: (i, 0)),
            pl.BlockSpec((blk, D), lambda i: (i, 0)),
        ],
        out_shape=[
            jax.ShapeDtypeStruct((N_NODES, D), jnp.float32),
            jax.ShapeDtypeStruct((N_NODES, D), jnp.float32),
        ],
    )(x, wd, ws)


EBLK_G = 40
NBLK_G = EDGES_PER_WORKER // EBLK_G
NSLOT = 3
NIT_G = 41


def _gather_kernel(p_hbm, q_hbm, dst3_hbm, src3_hbm, pd_hbm, qs_hbm,
                   dibuf, sibuf,
                   pa0, pa1, pa2, pb0, pb1, pb2,
                   qa0, qa1, qa2, qb0, qb1, qb2,
                   gsem, wsem):
    c = lax.axis_index("c")
    s = lax.axis_index("s")
    wid = s * NC + c
    ebase = wid * EDGES_PER_WORKER
    pA, pB = (pa0, pa1, pa2), (pb0, pb1, pb2)
    qA, qB = (qa0, qa1, qa2), (qb0, qb1, qb2)

    pltpu.sync_copy(dst3_hbm.at[wid], dibuf)
    pltpu.sync_copy(src3_hbm.at[wid], sibuf)

    def drain_writes(bufs):
        for b in bufs:
            pltpu.make_async_copy(b, pd_hbm.at[pl.ds(0, EBLK_G)], wsem).wait()

    def fire_gathers(j0, pset, qset):
        for b in range(NSLOT):
            pltpu.async_copy(p_hbm.at[dibuf.at[j0 + b]], pset[b], gsem)
            pltpu.async_copy(q_hbm.at[sibuf.at[j0 + b]], qset[b], gsem)

    def wait_gathers(pset, qset):
        for b in range(NSLOT):
            pltpu.make_async_copy(p_hbm.at[pl.ds(0, EBLK_G)], pset[b], gsem).wait()
            pltpu.make_async_copy(q_hbm.at[pl.ds(0, EBLK_G)], qset[b], gsem).wait()

    def fire_writes(j0, pset, qset):
        for b in range(NSLOT):
            off = pl.multiple_of(ebase + (j0 + b) * EBLK_G, 8)
            pltpu.async_copy(pset[b], pd_hbm.at[pl.ds(off, EBLK_G)], wsem)
            pltpu.async_copy(qset[b], qs_hbm.at[pl.ds(off, EBLK_G)], wsem)

    @pl.loop(0, NIT_G)
    def _(t):
        j0 = t * 2 * NSLOT

        @pl.when(t > 0)
        def _():
            drain_writes(pA)
            drain_writes(qA)
        fire_gathers(j0, pA, qA)

        @pl.when(t > 0)
        def _():
            drain_writes(pB)
            drain_writes(qB)
        fire_gathers(j0 + NSLOT, pB, qB)

        wait_gathers(pA, qA)
        fire_writes(j0, pA, qA)
        wait_gathers(pB, qB)
        fire_writes(j0 + NSLOT, pB, qB)

    drain_writes(pA)
    drain_writes(qA)
    drain_writes(pB)
    drain_writes(qB)

    for j, pbuf, qbuf in ((NIT_G * 2 * NSLOT, pA[0], qA[0]),
                          (NIT_G * 2 * NSLOT + 1, pA[1], qA[1]),
                          (NIT_G * 2 * NSLOT + 2, pA[2], qA[2]),
                          (NIT_G * 2 * NSLOT + 3, pB[0], qB[0])):
        off = pl.multiple_of(ebase + j * EBLK_G, 8)
        cp1 = pltpu.async_copy(p_hbm.at[dibuf.at[j]], pbuf, gsem)
        cp2 = pltpu.async_copy(q_hbm.at[sibuf.at[j]], qbuf, gsem)
        cp1.wait()
        cp2.wait()
        pltpu.sync_copy(pbuf, pd_hbm.at[pl.ds(off, EBLK_G)])
        pltpu.sync_copy(qbuf, qs_hbm.at[pl.ds(off, EBLK_G)])


def _gather_call(p, q, dst3, src3):
    mesh = plsc.VectorSubcoreMesh(core_axis_name="c", subcore_axis_name="s")
    rowbufs = [pltpu.VMEM((EBLK_G, D), jnp.float32)] * (4 * NSLOT)
    f = pl.kernel(
        _gather_kernel,
        mesh=mesh,
        out_type=(
            jax.ShapeDtypeStruct((N_EDGES, D), jnp.float32),
            jax.ShapeDtypeStruct((N_EDGES, D), jnp.float32),
        ),
        scratch_types=[
            pltpu.VMEM((NBLK_G, EBLK_G), jnp.int32),
            pltpu.VMEM((NBLK_G, EBLK_G), jnp.int32),
            *rowbufs,
            pltpu.SemaphoreType.DMA,
            pltpu.SemaphoreType.DMA,
        ],
    )
    return f(p, q, dst3, src3)


def _mlp_body(pd_ref, qs_ref, ea_ref, w1e_ref, b1_ref, g_ref, bt_ref, gm_ref,
              a_ref):
    h = (pd_ref[...] + qs_ref[...]
         + jnp.dot(ea_ref[...], w1e_ref[...], precision=_HI) + b1_ref[...])
    gm = gm_ref[...]
    m = jnp.dot(h, gm)
    sq = jnp.dot(h * h, gm)
    var = sq - m * m
    y = (h - m) * lax.rsqrt(var + EPS) * g_ref[...] + bt_ref[...]
    a_ref[...] = y * jax.nn.sigmoid(y)


def _mlp_call(pd, qs, ea, w1e, b1, gamma, beta, gmat):
    blk = 3200
    grid = N_EDGES // blk
    return pl.pallas_call(
        _mlp_body,
        grid=(grid,),
        in_specs=[
            pl.BlockSpec((blk, D), lambda i: (i, 0)),
            pl.BlockSpec((blk, D), lambda i: (i, 0)),
            pl.BlockSpec((blk, 4), lambda i: (i, 0)),
            pl.BlockSpec((4, D), lambda i: (0, 0)),
            pl.BlockSpec((1, D), lambda i: (0, 0)),
            pl.BlockSpec((1, D), lambda i: (0, 0)),
            pl.BlockSpec((1, D), lambda i: (0, 0)),
            pl.BlockSpec((D, D), lambda i: (0, 0)),
        ],
        out_specs=pl.BlockSpec((blk, D), lambda i: (i, 0)),
        out_shape=jax.ShapeDtypeStruct((N_EDGES, D), jnp.float32),
    )(pd, qs, ea, w1e, b1, gamma, beta, gmat)


def _count_kernel(dst3_hbm, zrow_hbm, ones_hbm, cp_hbm,
                  idxbuf, onesbuf, s_sh, csem):
    c = lax.axis_index("c")
    s = lax.axis_index("s")
    wid = s * NC + c
    rbase = pl.multiple_of(s * ROWS_PER_SUB, 8)
    rows = s_sh.at[pl.ds(rbase, ROWS_PER_SUB)]

    pltpu.sync_copy(dst3_hbm.at[wid], idxbuf)
    pltpu.sync_copy(ones_hbm, onesbuf)
    pltpu.sync_copy(zrow_hbm, rows)
    plsc.subcore_barrier()

    @pl.loop(0, NBLK)
    def _(j):
        pltpu.async_copy(onesbuf, s_sh.at[idxbuf.at[j]], csem, add=True)

    @pl.loop(0, NBLK)
    def _(j):
        pltpu.make_async_copy(onesbuf, s_sh.at[idxbuf.at[0]], csem).wait()

    plsc.subcore_barrier()
    pltpu.sync_copy(rows, cp_hbm.at[c, pl.ds(rbase, ROWS_PER_SUB)])


def _count_call(dst3, zrow, ones):
    mesh = plsc.VectorSubcoreMesh(core_axis_name="c", subcore_axis_name="s")
    f = pl.kernel(
        _count_kernel,
        mesh=mesh,
        out_type=jax.ShapeDtypeStruct((NC, N_PAD, D), jnp.float32),
        scratch_types=[
            pltpu.VMEM((NBLK, EBLK), jnp.int32),
            pltpu.VMEM((EBLK, D), jnp.float32),
            pltpu.VMEM_SHARED((N_PAD, D), jnp.float32),
            pltpu.SemaphoreType.DMA,
        ],
    )
    return f(dst3, zrow, ones)


def _scatter_kernel(a_hbm, dst3_hbm, zrow_hbm,
                    sp_hbm,
                    idxbuf, abuf0, abuf1, s_sh, asem0, asem1):
    c = lax.axis_index("c")
    s = lax.axis_index("s")
    wid = s * NC + c
    ebase = wid * EDGES_PER_WORKER
    rbase = pl.multiple_of(s * ROWS_PER_SUB, 8)
    rows = s_sh.at[pl.ds(rbase, ROWS_PER_SUB)]

    pltpu.sync_copy(dst3_hbm.at[wid], idxbuf)
    pltpu.sync_copy(zrow_hbm, rows)
    plsc.subcore_barrier()

    off0 = pl.multiple_of(ebase, 8)
    pltpu.sync_copy(a_hbm.at[pl.ds(off0, EBLK)], abuf0)

    @pl.loop(0, (NBLK - 1) // 2)
    def _(t):
        j0 = 2 * t
        add0 = pltpu.async_copy(abuf0, s_sh.at[idxbuf.at[j0]], asem0, add=True)
        off1 = pl.multiple_of(ebase + (j0 + 1) * EBLK, 8)
        pltpu.sync_copy(a_hbm.at[pl.ds(off1, EBLK)], abuf1)
        add0.wait()
        add1 = pltpu.async_copy(abuf1, s_sh.at[idxbuf.at[j0 + 1]], asem1, add=True)
        off2 = pl.multiple_of(ebase + (j0 + 2) * EBLK, 8)
        pltpu.sync_copy(a_hbm.at[pl.ds(off2, EBLK)], abuf0)
        add1.wait()

    pltpu.sync_copy(abuf0, s_sh.at[idxbuf.at[NBLK - 1]], add=True)

    plsc.subcore_barrier()
    pltpu.sync_copy(rows, sp_hbm.at[c, pl.ds(rbase, ROWS_PER_SUB)])


def _scatter_call(a, dst3, zrow):
    mesh = plsc.VectorSubcoreMesh(core_axis_name="c", subcore_axis_name="s")
    f = pl.kernel(
        _scatter_kernel,
        mesh=mesh,
        out_type=jax.ShapeDtypeStruct((NC, N_PAD, D), jnp.float32),
        scratch_types=[
            pltpu.VMEM((NBLK, EBLK), jnp.int32),
            pltpu.VMEM((EBLK, D), jnp.float32),
            pltpu.VMEM((EBLK, D), jnp.float32),
            pltpu.VMEM_SHARED((N_PAD, D), jnp.float32),
            pltpu.SemaphoreType.DMA,
            pltpu.SemaphoreType.DMA,
        ],
    )
    return f(a, dst3, zrow)


def _out_body(sp_ref, cp_ref, w2_ref, b2_ref, o_ref):
    sv = sp_ref[0] + sp_ref[1]
    cnt = cp_ref[0, :, 0:1] + cp_ref[1, :, 0:1]
    msg = jnp.dot(sv, w2_ref[...], precision=_HI) + cnt * b2_ref[...]
    o_ref[...] = msg / jnp.maximum(cnt, 1.0)


def _out_call(sp, cp, w2, b2):
    blk = 400
    grid = N_NODES // blk
    return pl.pallas_call(
        _out_body,
        grid=(grid,),
        in_specs=[
            pl.BlockSpec((NC, blk, D), lambda i: (0, i, 0)),
            pl.BlockSpec((NC, blk, D), lambda i: (0, i, 0)),
            pl.BlockSpec((D, D), lambda i: (0, 0)),
            pl.BlockSpec((1, D), lambda i: (0, 0)),
        ],
        out_specs=pl.BlockSpec((blk, D), lambda i: (i, 0)),
        out_shape=jax.ShapeDtypeStruct((N_NODES, D), jnp.float32),
    )(sp, cp, w2, b2)


def kernel(x, edge_index, edge_attr, W1, b1, gamma, beta, W2, b2):
    src = edge_index[0]
    dst = edge_index[1]
    w1d = W1[0:D]
    w1s = W1[D:2 * D]
    w1e = W1[2 * D:]
    gmat = jnp.asarray(_GM_NP)
    zrow = jnp.zeros((ROWS_PER_SUB, D), jnp.float32)
    ones = jnp.ones((EBLK, D), jnp.float32)

    p, q = _pq_call(x, w1d, w1s)
    dst3 = dst.reshape(NW, NBLK_G, EBLK_G)
    src3 = src.reshape(NW, NBLK_G, EBLK_G)
    pd, qs = _gather_call(p, q, dst3, src3)
    a = _mlp_call(pd, qs, edge_attr, w1e,
                  b1.reshape(1, D), gamma.reshape(1, D), beta.reshape(1, D),
                  gmat)
    dstS = dst.reshape(NW, NBLK, EBLK)
    cp = _count_call(dstS, zrow, ones)
    sp = _scatter_call(a, dstS, zrow)
    return _out_call(sp, cp, W2, b2.reshape(1, D))


# speedup vs baseline: 4.6255x; 1.0354x over previous
import functools

import jax
import jax.numpy as jnp
import numpy as np
from jax import lax
from jax.experimental import pallas as pl
from jax.experimental.pallas import tpu as pltpu
from jax.experimental.pallas import tpu_sc as plsc

N_NODES = 10000
N_EDGES = 320000
D = 128
N_GROUPS = 8
GROUP_SIZE = 16
EPS = 1e-5

NC = 2
NS = 16
NW = NC * NS
EDGES_PER_WORKER = N_EDGES // NW
EBLK = 80
NBLK = EDGES_PER_WORKER // EBLK
N_PAD = 10240
ROWS_PER_SUB = N_PAD // NS

_HI = lax.Precision.HIGHEST

_GM_NP = np.kron(np.eye(N_GROUPS, dtype=np.float32),
                 np.ones((GROUP_SIZE, GROUP_SIZE), dtype=np.float32)) / GROUP_SIZE


def _pq_body(x_ref, wd_ref, ws_ref, p_ref, q_ref):
    xv = x_ref[...]
    p_ref[...] = jnp.dot(xv, wd_ref[...], precision=_HI)
    q_ref[...] = jnp.dot(xv, ws_ref[...], precision=_HI)


def _pq_call(x, wd, ws):
    blk = 400
    grid = N_NODES // blk
    return pl.pallas_call(
        _pq_body,
        grid=(grid,),
        in_specs=[
            pl.BlockSpec((blk, D), lambda i: (i, 0)),
            pl.BlockSpec((D, D), lambda i: (0, 0)),
            pl.BlockSpec((D, D), lambda i: (0, 0)),
        ],
        out_specs=[
            pl.BlockSpec((blk, D), lambda i: (i, 0)),
            pl.BlockSpec((blk, D), lambda i: (i, 0)),
        ],
        out_shape=[
            jax.ShapeDtypeStruct((N_NODES, D), jnp.float32),
            jax.ShapeDtypeStruct((N_NODES, D), jnp.float32),
        ],
    )(x, wd, ws)


EBLK_G = 40
NSLOT = 3


def _make_gather_kernel(epw, nblk, ch_base, n_ch_edges):
    nit = nblk // (2 * NSLOT)
    rem = nblk - nit * 2 * NSLOT

    def _gather_kernel(p_hbm, q_hbm, dst3_hbm, src3_hbm, pd_hbm, qs_hbm,
                       dibuf, sibuf,
                       pa0, pa1, pa2, pb0, pb1, pb2,
                       qa0, qa1, qa2, qb0, qb1, qb2,
                       gsem, wsem):
        c = lax.axis_index("c")
        s = lax.axis_index("s")
        wid = s * NC + c
        ebase = wid * epw
        pA, pB = (pa0, pa1, pa2), (pb0, pb1, pb2)
        qA, qB = (qa0, qa1, qa2), (qb0, qb1, qb2)

        pltpu.sync_copy(dst3_hbm.at[wid], dibuf)
        pltpu.sync_copy(src3_hbm.at[wid], sibuf)

        def drain_writes(bufs):
            for b in bufs:
                pltpu.make_async_copy(b, pd_hbm.at[pl.ds(0, EBLK_G)], wsem).wait()

        def fire_gathers(j0, pset, qset):
            for b in range(NSLOT):
                pltpu.async_copy(p_hbm.at[dibuf.at[j0 + b]], pset[b], gsem)
                pltpu.async_copy(q_hbm.at[sibuf.at[j0 + b]], qset[b], gsem)

        def wait_gathers(pset, qset):
            for b in range(NSLOT):
                pltpu.make_async_copy(p_hbm.at[pl.ds(0, EBLK_G)], pset[b], gsem).wait()
                pltpu.make_async_copy(q_hbm.at[pl.ds(0, EBLK_G)], qset[b], gsem).wait()

        def fire_writes(j0, pset, qset):
            for b in range(NSLOT):
                off = pl.multiple_of(ebase + (j0 + b) * EBLK_G, 8)
                pltpu.async_copy(pset[b], pd_hbm.at[pl.ds(off, EBLK_G)], wsem)
                pltpu.async_copy(qset[b], qs_hbm.at[pl.ds(off, EBLK_G)], wsem)

        @pl.loop(0, nit)
        def _(t):
            j0 = t * 2 * NSLOT

            @pl.when(t > 0)
            def _():
                drain_writes(pA)
                drain_writes(qA)
            fire_gathers(j0, pA, qA)

            @pl.when(t > 0)
            def _():
                drain_writes(pB)
                drain_writes(qB)
            fire_gathers(j0 + NSLOT, pB, qB)

            wait_gathers(pA, qA)
            fire_writes(j0, pA, qA)
            wait_gathers(pB, qB)
            fire_writes(j0 + NSLOT, pB, qB)

        drain_writes(pA)
        drain_writes(qA)
        drain_writes(pB)
        drain_writes(qB)

        tail_bufs = [(pa0, qa0), (pa1, qa1), (pa2, qa2), (pb0, qb0), (pb1, qb1)]
        for i in range(rem):
            j = nit * 2 * NSLOT + i
            pbuf, qbuf = tail_bufs[i]
            off = pl.multiple_of(ebase + j * EBLK_G, 8)
            cp1 = pltpu.async_copy(p_hbm.at[dibuf.at[j]], pbuf, gsem)
            cp2 = pltpu.async_copy(q_hbm.at[sibuf.at[j]], qbuf, gsem)
            cp1.wait()
            cp2.wait()
            pltpu.sync_copy(pbuf, pd_hbm.at[pl.ds(off, EBLK_G)])
            pltpu.sync_copy(qbuf, qs_hbm.at[pl.ds(off, EBLK_G)])

    return _gather_kernel


def _gather_call(p, q, dst3, src3, epw, n_ch_edges):
    nblk = epw // EBLK_G
    mesh = plsc.VectorSubcoreMesh(core_axis_name="c", subcore_axis_name="s")
    rowbufs = [pltpu.VMEM((EBLK_G, D), jnp.float32)] * (4 * NSLOT)
    f = pl.kernel(
        _make_gather_kernel(epw, nblk, 0, n_ch_edges),
        mesh=mesh,
        out_type=(
            jax.ShapeDtypeStruct((n_ch_edges, D), jnp.float32),
            jax.ShapeDtypeStruct((n_ch_edges, D), jnp.float32),
        ),
        scratch_types=[
            pltpu.VMEM((nblk, EBLK_G), jnp.int32),
            pltpu.VMEM((nblk, EBLK_G), jnp.int32),
            *rowbufs,
            pltpu.SemaphoreType.DMA,
            pltpu.SemaphoreType.DMA,
        ],
    )
    return f(p, q, dst3, src3)


def _mlp_body(pd_ref, qs_ref, ea_ref, w1e_ref, b1_ref, g_ref, bt_ref, gm_ref,
              a_ref):
    h = (pd_ref[...] + qs_ref[...]
         + jnp.dot(ea_ref[...], w1e_ref[...], precision=_HI) + b1_ref[...])
    gm = gm_ref[...]
    m = jnp.dot(h, gm)
    sq = jnp.dot(h * h, gm)
    var = sq - m * m
    y = (h - m) * lax.rsqrt(var + EPS) * g_ref[...] + bt_ref[...]
    a_ref[...] = y * jax.nn.sigmoid(y)


def _mlp_call(pd, qs, ea, w1e, b1, gamma, beta, gmat):
    blk = 3200
    ne = pd.shape[0]
    grid = ne // blk
    return pl.pallas_call(
        _mlp_body,
        grid=(grid,),
        in_specs=[
            pl.BlockSpec((blk, D), lambda i: (i, 0)),
            pl.BlockSpec((blk, D), lambda i: (i, 0)),
            pl.BlockSpec((blk, 4), lambda i: (i, 0)),
            pl.BlockSpec((4, D), lambda i: (0, 0)),
            pl.BlockSpec((1, D), lambda i: (0, 0)),
            pl.BlockSpec((1, D), lambda i: (0, 0)),
            pl.BlockSpec((1, D), lambda i: (0, 0)),
            pl.BlockSpec((D, D), lambda i: (0, 0)),
        ],
        out_specs=pl.BlockSpec((blk, D), lambda i: (i, 0)),
        out_shape=jax.ShapeDtypeStruct((ne, D), jnp.float32),
    )(pd, qs, ea, w1e, b1, gamma, beta, gmat)


def _count_kernel(dst3_hbm, zrow_hbm, ones_hbm, cp_hbm,
                  idxbuf, onesbuf, s_sh, csem):
    c = lax.axis_index("c")
    s = lax.axis_index("s")
    wid = s * NC + c
    rbase = pl.multiple_of(s * ROWS_PER_SUB, 8)
    rows = s_sh.at[pl.ds(rbase, ROWS_PER_SUB)]

    pltpu.sync_copy(dst3_hbm.at[wid], idxbuf)
    pltpu.sync_copy(ones_hbm, onesbuf)
    pltpu.sync_copy(zrow_hbm, rows)
    plsc.subcore_barrier()

    @pl.loop(0, NBLK)
    def _(j):
        pltpu.async_copy(onesbuf, s_sh.at[idxbuf.at[j]], csem, add=True)

    @pl.loop(0, NBLK)
    def _(j):
        pltpu.make_async_copy(onesbuf, s_sh.at[idxbuf.at[0]], csem).wait()

    plsc.subcore_barrier()
    pltpu.sync_copy(rows, cp_hbm.at[c, pl.ds(rbase, ROWS_PER_SUB)])


def _count_call(dst3, zrow, ones):
    mesh = plsc.VectorSubcoreMesh(core_axis_name="c", subcore_axis_name="s")
    f = pl.kernel(
        _count_kernel,
        mesh=mesh,
        out_type=jax.ShapeDtypeStruct((NC, N_PAD, D), jnp.float32),
        scratch_types=[
            pltpu.VMEM((NBLK, EBLK), jnp.int32),
            pltpu.VMEM((EBLK, D), jnp.float32),
            pltpu.VMEM_SHARED((N_PAD, D), jnp.float32),
            pltpu.SemaphoreType.DMA,
        ],
    )
    return f(dst3, zrow, ones)


EBLK_S = 40


def _make_scatter_kernel(epw, nblk):
    def _scatter_kernel(a_hbm, dst3_hbm, zrow_hbm,
                        sp_hbm,
                        idxbuf, abuf0, abuf1, s_sh, asem0, asem1):
        c = lax.axis_index("c")
        s = lax.axis_index("s")
        wid = s * NC + c
        ebase = wid * epw
        rbase = pl.multiple_of(s * ROWS_PER_SUB, 8)
        rows = s_sh.at[pl.ds(rbase, ROWS_PER_SUB)]

        pltpu.sync_copy(dst3_hbm.at[wid], idxbuf)
        pltpu.sync_copy(zrow_hbm, rows)
        plsc.subcore_barrier()

        off0 = pl.multiple_of(ebase, 8)
        pltpu.sync_copy(a_hbm.at[pl.ds(off0, EBLK_S)], abuf0)

        @pl.loop(0, (nblk - 1) // 2)
        def _(t):
            j0 = 2 * t
            add0 = pltpu.async_copy(abuf0, s_sh.at[idxbuf.at[j0]], asem0, add=True)
            off1 = pl.multiple_of(ebase + (j0 + 1) * EBLK_S, 8)
            pltpu.sync_copy(a_hbm.at[pl.ds(off1, EBLK_S)], abuf1)
            add0.wait()
            add1 = pltpu.async_copy(abuf1, s_sh.at[idxbuf.at[j0 + 1]], asem1, add=True)
            off2 = pl.multiple_of(ebase + (j0 + 2) * EBLK_S, 8)
            pltpu.sync_copy(a_hbm.at[pl.ds(off2, EBLK_S)], abuf0)
            add1.wait()

        pltpu.sync_copy(abuf0, s_sh.at[idxbuf.at[nblk - 1]], add=True)

        plsc.subcore_barrier()
        pltpu.sync_copy(rows, sp_hbm.at[c, pl.ds(rbase, ROWS_PER_SUB)])

    return _scatter_kernel


def _scatter_call(a, dst3, zrow, epw):
    nblk = epw // EBLK_S
    mesh = plsc.VectorSubcoreMesh(core_axis_name="c", subcore_axis_name="s")
    f = pl.kernel(
        _make_scatter_kernel(epw, nblk),
        mesh=mesh,
        out_type=jax.ShapeDtypeStruct((NC, N_PAD, D), jnp.float32),
        scratch_types=[
            pltpu.VMEM((nblk, EBLK_S), jnp.int32),
            pltpu.VMEM((EBLK_S, D), jnp.float32),
            pltpu.VMEM((EBLK_S, D), jnp.float32),
            pltpu.VMEM_SHARED((N_PAD, D), jnp.float32),
            pltpu.SemaphoreType.DMA,
            pltpu.SemaphoreType.DMA,
        ],
    )
    return f(a, dst3, zrow)


def _out_body(sp_ref, sq_ref, cp_ref, w2_ref, b2_ref, o_ref):
    sv = sp_ref[0] + sp_ref[1] + sq_ref[0] + sq_ref[1]
    cnt = cp_ref[0, :, 0:1] + cp_ref[1, :, 0:1]
    msg = jnp.dot(sv, w2_ref[...], precision=_HI) + cnt * b2_ref[...]
    o_ref[...] = msg / jnp.maximum(cnt, 1.0)


def _out_call(sp, sq, cp, w2, b2):
    blk = 400
    grid = N_NODES // blk
    return pl.pallas_call(
        _out_body,
        grid=(grid,),
        in_specs=[
            pl.BlockSpec((NC, blk, D), lambda i: (0, i, 0)),
            pl.BlockSpec((NC, blk, D), lambda i: (0, i, 0)),
            pl.BlockSpec((NC, blk, D), lambda i: (0, i, 0)),
            pl.BlockSpec((D, D), lambda i: (0, 0)),
            pl.BlockSpec((1, D), lambda i: (0, 0)),
        ],
        out_specs=pl.BlockSpec((blk, D), lambda i: (i, 0)),
        out_shape=jax.ShapeDtypeStruct((N_NODES, D), jnp.float32),
    )(sp, sq, cp, w2, b2)


def kernel(x, edge_index, edge_attr, W1, b1, gamma, beta, W2, b2):
    src = edge_index[0]
    dst = edge_index[1]
    w1d = W1[0:D]
    w1s = W1[D:2 * D]
    w1e = W1[2 * D:]
    gmat = jnp.asarray(_GM_NP)
    zrow = jnp.zeros((ROWS_PER_SUB, D), jnp.float32)
    ones = jnp.ones((EBLK, D), jnp.float32)

    p, q = _pq_call(x, w1d, w1s)

    ech = N_EDGES // 2
    epw = ech // NW
    nblk_g = epw // EBLK_G
    nblk_s = epw // EBLK_S

    sps = []
    for k in range(2):
        dk = lax.dynamic_slice_in_dim(dst, k * ech, ech)
        sk = lax.dynamic_slice_in_dim(src, k * ech, ech)
        eak = lax.dynamic_slice_in_dim(edge_attr, k * ech, ech)
        dg = dk.reshape(NW, nblk_g, EBLK_G)
        sg = sk.reshape(NW, nblk_g, EBLK_G)
        pd, qs = _gather_call(p, q, dg, sg, epw, ech)
        a = _mlp_call(pd, qs, eak, w1e,
                      b1.reshape(1, D), gamma.reshape(1, D), beta.reshape(1, D),
                      gmat)
        ds_ = dk.reshape(NW, nblk_s, EBLK_S)
        sps.append(_scatter_call(a, ds_, zrow, epw))

    dstC = dst.reshape(NW, NBLK, EBLK)
    cp = _count_call(dstC, zrow, ones)
    return _out_call(sps[0], sps[1], cp, W2, b2.reshape(1, D))


# speedup vs baseline: 4.9097x; 1.0614x over previous
import functools

import jax
import jax.numpy as jnp
import numpy as np
from jax import lax
from jax.experimental import pallas as pl
from jax.experimental.pallas import tpu as pltpu
from jax.experimental.pallas import tpu_sc as plsc

N_NODES = 10000
N_EDGES = 320000
D = 128
N_GROUPS = 8
GROUP_SIZE = 16
EPS = 1e-5

NC = 2
NS = 16
NW = NC * NS
EDGES_PER_WORKER = N_EDGES // NW
EBLK = 80
NBLK = EDGES_PER_WORKER // EBLK
N_PAD = 10240
ROWS_PER_SUB = N_PAD // NS

_HI = lax.Precision.HIGHEST

_GM_NP = np.kron(np.eye(N_GROUPS, dtype=np.float32),
                 np.ones((GROUP_SIZE, GROUP_SIZE), dtype=np.float32)) / GROUP_SIZE


def _pq_body(x_ref, wd_ref, ws_ref, p_ref, q_ref):
    xv = x_ref[...]
    p_ref[...] = jnp.dot(xv, wd_ref[...], precision=_HI)
    q_ref[...] = jnp.dot(xv, ws_ref[...], precision=_HI)


def _pq_call(x, wd, ws):
    blk = 400
    grid = N_NODES // blk
    return pl.pallas_call(
        _pq_body,
        grid=(grid,),
        in_specs=[
            pl.BlockSpec((blk, D), lambda i: (i, 0)),
            pl.BlockSpec((D, D), lambda i: (0, 0)),
            pl.BlockSpec((D, D), lambda i: (0, 0)),
        ],
        out_specs=[
            pl.BlockSpec((blk, D), lambda i: (i, 0)),
            pl.BlockSpec((blk, D), lambda i: (i, 0)),
        ],
        out_shape=[
            jax.ShapeDtypeStruct((N_NODES, D), jnp.float32),
            jax.ShapeDtypeStruct((N_NODES, D), jnp.float32),
        ],
    )(x, wd, ws)


EBLK_G = 40
NSLOT = 3


def _make_gather_kernel(epw, nblk, ch_base, n_ch_edges):
    nit = nblk // (2 * NSLOT)
    rem = nblk - nit * 2 * NSLOT

    def _gather_kernel(p_hbm, q_hbm, dst3_hbm, src3_hbm, pd_hbm, qs_hbm,
                       dibuf, sibuf,
                       pa0, pa1, pa2, pb0, pb1, pb2,
                       qa0, qa1, qa2, qb0, qb1, qb2,
                       gsem, wsem):
        c = lax.axis_index("c")
        s = lax.axis_index("s")
        wid = s * NC + c
        ebase = wid * epw
        pA, pB = (pa0, pa1, pa2), (pb0, pb1, pb2)
        qA, qB = (qa0, qa1, qa2), (qb0, qb1, qb2)

        pltpu.sync_copy(dst3_hbm.at[wid], dibuf)
        pltpu.sync_copy(src3_hbm.at[wid], sibuf)

        def drain_writes(bufs):
            for b in bufs:
                pltpu.make_async_copy(b, pd_hbm.at[pl.ds(0, EBLK_G)], wsem).wait()

        def fire_gathers(j0, pset, qset):
            for b in range(NSLOT):
                pltpu.async_copy(p_hbm.at[dibuf.at[j0 + b]], pset[b], gsem)
                pltpu.async_copy(q_hbm.at[sibuf.at[j0 + b]], qset[b], gsem)

        def wait_gathers(pset, qset):
            for b in range(NSLOT):
                pltpu.make_async_copy(p_hbm.at[pl.ds(0, EBLK_G)], pset[b], gsem).wait()
                pltpu.make_async_copy(q_hbm.at[pl.ds(0, EBLK_G)], qset[b], gsem).wait()

        def fire_writes(j0, pset, qset):
            for b in range(NSLOT):
                off = pl.multiple_of(ebase + (j0 + b) * EBLK_G, 8)
                pltpu.async_copy(pset[b], pd_hbm.at[pl.ds(off, EBLK_G)], wsem)
                pltpu.async_copy(qset[b], qs_hbm.at[pl.ds(off, EBLK_G)], wsem)

        @pl.loop(0, nit)
        def _(t):
            j0 = t * 2 * NSLOT

            @pl.when(t > 0)
            def _():
                drain_writes(pA)
                drain_writes(qA)
            fire_gathers(j0, pA, qA)

            @pl.when(t > 0)
            def _():
                drain_writes(pB)
                drain_writes(qB)
            fire_gathers(j0 + NSLOT, pB, qB)

            wait_gathers(pA, qA)
            fire_writes(j0, pA, qA)
            wait_gathers(pB, qB)
            fire_writes(j0 + NSLOT, pB, qB)

        drain_writes(pA)
        drain_writes(qA)
        drain_writes(pB)
        drain_writes(qB)

        tail_bufs = [(pa0, qa0), (pa1, qa1), (pa2, qa2), (pb0, qb0), (pb1, qb1)]
        for i in range(rem):
            j = nit * 2 * NSLOT + i
            pbuf, qbuf = tail_bufs[i]
            off = pl.multiple_of(ebase + j * EBLK_G, 8)
            cp1 = pltpu.async_copy(p_hbm.at[dibuf.at[j]], pbuf, gsem)
            cp2 = pltpu.async_copy(q_hbm.at[sibuf.at[j]], qbuf, gsem)
            cp1.wait()
            cp2.wait()
            pltpu.sync_copy(pbuf, pd_hbm.at[pl.ds(off, EBLK_G)])
            pltpu.sync_copy(qbuf, qs_hbm.at[pl.ds(off, EBLK_G)])

    return _gather_kernel


def _gather_call(p, q, dst3, src3, epw, n_ch_edges):
    nblk = epw // EBLK_G
    mesh = plsc.VectorSubcoreMesh(core_axis_name="c", subcore_axis_name="s")
    rowbufs = [pltpu.VMEM((EBLK_G, D), jnp.float32)] * (4 * NSLOT)
    f = pl.kernel(
        _make_gather_kernel(epw, nblk, 0, n_ch_edges),
        mesh=mesh,
        out_type=(
            jax.ShapeDtypeStruct((n_ch_edges, D), jnp.float32),
            jax.ShapeDtypeStruct((n_ch_edges, D), jnp.float32),
        ),
        scratch_types=[
            pltpu.VMEM((nblk, EBLK_G), jnp.int32),
            pltpu.VMEM((nblk, EBLK_G), jnp.int32),
            *rowbufs,
            pltpu.SemaphoreType.DMA,
            pltpu.SemaphoreType.DMA,
        ],
    )
    return f(p, q, dst3, src3)


def _mlp_body(pd_ref, qs_ref, ea_ref, w1e_ref, b1_ref, g_ref, bt_ref, gm_ref,
              a_ref):
    h = (pd_ref[...] + qs_ref[...]
         + jnp.dot(ea_ref[...], w1e_ref[...], precision=_HI) + b1_ref[...])
    gm = gm_ref[...].astype(jnp.bfloat16)
    m = jnp.dot(h.astype(jnp.bfloat16), gm, preferred_element_type=jnp.float32)
    sq = jnp.dot((h * h).astype(jnp.bfloat16), gm, preferred_element_type=jnp.float32)
    var = sq - m * m
    y = (h - m) * lax.rsqrt(var + EPS) * g_ref[...] + bt_ref[...]
    a_ref[...] = y * jax.nn.sigmoid(y)


def _mlp_call(pd, qs, ea, w1e, b1, gamma, beta, gmat):
    blk = 3200
    ne = pd.shape[0]
    grid = ne // blk
    return pl.pallas_call(
        _mlp_body,
        grid=(grid,),
        in_specs=[
            pl.BlockSpec((blk, D), lambda i: (i, 0)),
            pl.BlockSpec((blk, D), lambda i: (i, 0)),
            pl.BlockSpec((blk, 4), lambda i: (i, 0)),
            pl.BlockSpec((4, D), lambda i: (0, 0)),
            pl.BlockSpec((1, D), lambda i: (0, 0)),
            pl.BlockSpec((1, D), lambda i: (0, 0)),
            pl.BlockSpec((1, D), lambda i: (0, 0)),
            pl.BlockSpec((D, D), lambda i: (0, 0)),
        ],
        out_specs=pl.BlockSpec((blk, D), lambda i: (i, 0)),
        out_shape=jax.ShapeDtypeStruct((ne, D), jnp.float32),
    )(pd, qs, ea, w1e, b1, gamma, beta, gmat)


def _count_kernel(dst3_hbm, zrow_hbm, ones_hbm, cp_hbm,
                  idxbuf, onesbuf, s_sh, csem):
    c = lax.axis_index("c")
    s = lax.axis_index("s")
    wid = s * NC + c
    rbase = pl.multiple_of(s * ROWS_PER_SUB, 8)
    rows = s_sh.at[pl.ds(rbase, ROWS_PER_SUB)]

    pltpu.sync_copy(dst3_hbm.at[wid], idxbuf)
    pltpu.sync_copy(ones_hbm, onesbuf)
    pltpu.sync_copy(zrow_hbm, rows)
    plsc.subcore_barrier()

    @pl.loop(0, NBLK)
    def _(j):
        pltpu.async_copy(onesbuf, s_sh.at[idxbuf.at[j]], csem, add=True)

    @pl.loop(0, NBLK)
    def _(j):
        pltpu.make_async_copy(onesbuf, s_sh.at[idxbuf.at[0]], csem).wait()

    plsc.subcore_barrier()
    pltpu.sync_copy(rows, cp_hbm.at[c, pl.ds(rbase, ROWS_PER_SUB)])


def _count_call(dst3, zrow, ones):
    mesh = plsc.VectorSubcoreMesh(core_axis_name="c", subcore_axis_name="s")
    f = pl.kernel(
        _count_kernel,
        mesh=mesh,
        out_type=jax.ShapeDtypeStruct((NC, N_PAD, D), jnp.float32),
        scratch_types=[
            pltpu.VMEM((NBLK, EBLK), jnp.int32),
            pltpu.VMEM((EBLK, D), jnp.float32),
            pltpu.VMEM_SHARED((N_PAD, D), jnp.float32),
            pltpu.SemaphoreType.DMA,
        ],
    )
    return f(dst3, zrow, ones)


EBLK_S = 40


def _make_scatter_kernel(epw, nblk):
    nit = (nblk - 1) // 4
    rem = nblk - nit * 4

    def _scatter_kernel(a_hbm, dst3_hbm, zrow_hbm,
                        sp_hbm,
                        idxbuf, aa0, aa1, ab0, ab1, s_sh, lsem, asemA, asemB):
        c = lax.axis_index("c")
        s = lax.axis_index("s")
        wid = s * NC + c
        ebase = wid * epw
        rbase = pl.multiple_of(s * ROWS_PER_SUB, 8)
        rows = s_sh.at[pl.ds(rbase, ROWS_PER_SUB)]

        pltpu.sync_copy(dst3_hbm.at[wid], idxbuf)
        pltpu.sync_copy(zrow_hbm, rows)
        plsc.subcore_barrier()

        def fire_loads(j0, bufs):
            for k, b in enumerate(bufs):
                off = pl.multiple_of(ebase + (j0 + k) * EBLK_S, 8)
                pltpu.async_copy(a_hbm.at[pl.ds(off, EBLK_S)], b, lsem)

        def wait_loads(bufs):
            for b in bufs:
                pltpu.make_async_copy(a_hbm.at[pl.ds(0, EBLK_S)], b, lsem).wait()

        def fire_adds(j0, bufs, sem):
            for k, b in enumerate(bufs):
                pltpu.async_copy(b, s_sh.at[idxbuf.at[j0 + k]], sem, add=True)

        def drain_adds(bufs, sem):
            for b in bufs:
                pltpu.make_async_copy(b, s_sh.at[idxbuf.at[0]], sem).wait()

        A = (aa0, aa1)
        B = (ab0, ab1)

        @pl.loop(0, nit)
        def _(t):
            j0 = 4 * t

            @pl.when(t > 0)
            def _():
                drain_adds(A, asemA)
            fire_loads(j0, A)
            wait_loads(A)
            fire_adds(j0, A, asemA)

            @pl.when(t > 0)
            def _():
                drain_adds(B, asemB)
            fire_loads(j0 + 2, B)
            wait_loads(B)
            fire_adds(j0 + 2, B, asemB)

        drain_adds(A, asemA)
        drain_adds(B, asemB)
        for i in range(rem):
            j = nit * 4 + i
            off = pl.multiple_of(ebase + j * EBLK_S, 8)
            pltpu.sync_copy(a_hbm.at[pl.ds(off, EBLK_S)], A[0])
            pltpu.sync_copy(A[0], s_sh.at[idxbuf.at[j]], add=True)

        plsc.subcore_barrier()
        pltpu.sync_copy(rows, sp_hbm.at[c, pl.ds(rbase, ROWS_PER_SUB)])

    return _scatter_kernel


def _scatter_call(a, dst3, zrow, epw):
    nblk = epw // EBLK_S
    mesh = plsc.VectorSubcoreMesh(core_axis_name="c", subcore_axis_name="s")
    f = pl.kernel(
        _make_scatter_kernel(epw, nblk),
        mesh=mesh,
        out_type=jax.ShapeDtypeStruct((NC, N_PAD, D), jnp.float32),
        scratch_types=[
            pltpu.VMEM((nblk, EBLK_S), jnp.int32),
            pltpu.VMEM((EBLK_S, D), jnp.float32),
            pltpu.VMEM((EBLK_S, D), jnp.float32),
            pltpu.VMEM((EBLK_S, D), jnp.float32),
            pltpu.VMEM((EBLK_S, D), jnp.float32),
            pltpu.VMEM_SHARED((N_PAD, D), jnp.float32),
            pltpu.SemaphoreType.DMA,
            pltpu.SemaphoreType.DMA,
            pltpu.SemaphoreType.DMA,
        ],
    )
    return f(a, dst3, zrow)


def _out_body(sp_ref, sq_ref, cp_ref, w2_ref, b2_ref, o_ref):
    sv = sp_ref[0] + sp_ref[1] + sq_ref[0] + sq_ref[1]
    cnt = cp_ref[0, :, 0:1] + cp_ref[1, :, 0:1]
    msg = jnp.dot(sv, w2_ref[...], precision=_HI) + cnt * b2_ref[...]
    o_ref[...] = msg / jnp.maximum(cnt, 1.0)


def _out_call(sp, sq, cp, w2, b2):
    blk = 400
    grid = N_NODES // blk
    return pl.pallas_call(
        _out_body,
        grid=(grid,),
        in_specs=[
            pl.BlockSpec((NC, blk, D), lambda i: (0, i, 0)),
            pl.BlockSpec((NC, blk, D), lambda i: (0, i, 0)),
            pl.BlockSpec((NC, blk, D), lambda i: (0, i, 0)),
            pl.BlockSpec((D, D), lambda i: (0, 0)),
            pl.BlockSpec((1, D), lambda i: (0, 0)),
        ],
        out_specs=pl.BlockSpec((blk, D), lambda i: (i, 0)),
        out_shape=jax.ShapeDtypeStruct((N_NODES, D), jnp.float32),
    )(sp, sq, cp, w2, b2)


def kernel(x, edge_index, edge_attr, W1, b1, gamma, beta, W2, b2):
    src = edge_index[0]
    dst = edge_index[1]
    w1d = W1[0:D]
    w1s = W1[D:2 * D]
    w1e = W1[2 * D:]
    gmat = jnp.asarray(_GM_NP)
    zrow = jnp.zeros((ROWS_PER_SUB, D), jnp.float32)
    ones = jnp.ones((EBLK, D), jnp.float32)

    p, q = _pq_call(x, w1d, w1s)

    ech = N_EDGES // 2
    epw = ech // NW
    nblk_g = epw // EBLK_G
    nblk_s = epw // EBLK_S

    sps = []
    for k in range(2):
        dk = lax.dynamic_slice_in_dim(dst, k * ech, ech)
        sk = lax.dynamic_slice_in_dim(src, k * ech, ech)
        eak = lax.dynamic_slice_in_dim(edge_attr, k * ech, ech)
        dg = dk.reshape(NW, nblk_g, EBLK_G)
        sg = sk.reshape(NW, nblk_g, EBLK_G)
        pd, qs = _gather_call(p, q, dg, sg, epw, ech)
        a = _mlp_call(pd, qs, eak, w1e,
                      b1.reshape(1, D), gamma.reshape(1, D), beta.reshape(1, D),
                      gmat)
        ds_ = dk.reshape(NW, nblk_s, EBLK_S)
        sps.append(_scatter_call(a, ds_, zrow, epw))

    dstC = dst.reshape(NW, NBLK, EBLK)
    cp = _count_call(dstC, zrow, ones)
    return _out_call(sps[0], sps[1], cp, W2, b2.reshape(1, D))
